# Initial kernel scaffold; baseline (speedup 1.0000x reference)
import functools
import math

import jax
import jax.numpy as jnp
import numpy as np
from jax.experimental import pallas as pl
from jax.experimental.pallas import tpu as pltpu

B, S, D = 1, 2048, 768
H, KVH, HD = 12, 4, 64
E, TOPK, F = 8, 2, 2048
EPS, THETA = 1e-6, 10000.0
REP = H // KVH

SB = 512
NSB = S // SB
BQ = 512
BK = 512
NEG = -1e30


def _rms(x, w):
    var = jnp.mean(jnp.square(x), axis=-1, keepdims=True)
    return (x * jax.lax.rsqrt(var + EPS)) * w


def _rot_half_heads(x):
    half = HD // 2
    s = jnp.concatenate([x[:, half:], x[:, :half]], axis=1)
    t = jnp.concatenate([x[:, -half:], x[:, :-half]], axis=1)
    lane = jax.lax.broadcasted_iota(jnp.int32, x.shape, 1)
    first = (lane % HD) < half
    return jnp.where(first, -s, t)


def _qkv_kernel(x_ref, ln1_ref, wq_ref, wk_ref, wv_ref, cq_ref, sq_ref,
                ck_ref, sk_ref, q_ref, k_ref, v_ref):
    h = _rms(x_ref[...], ln1_ref[...])
    hb = h.astype(jnp.bfloat16)
    q = jnp.dot(hb, wq_ref[...], preferred_element_type=jnp.float32)
    k = jnp.dot(hb, wk_ref[...], preferred_element_type=jnp.float32)
    v = jnp.dot(hb, wv_ref[...], preferred_element_type=jnp.float32)
    q = q * cq_ref[...] + _rot_half_heads(q) * sq_ref[...]
    k = k * ck_ref[...] + _rot_half_heads(k) * sk_ref[...]
    q_ref[...] = q.astype(jnp.bfloat16)
    k_ref[...] = k.astype(jnp.bfloat16)
    v_ref[...] = v.astype(jnp.bfloat16)


def _qkv(x, ln1_w, wq, wk, wv, cos_q, sin_q, cos_k, sin_k):
    return pl.pallas_call(
        _qkv_kernel,
        grid=(NSB,),
        in_specs=[
            pl.BlockSpec((SB, D), lambda i: (i, 0)),
            pl.BlockSpec((1, D), lambda i: (0, 0)),
            pl.BlockSpec((D, H * HD), lambda i: (0, 0)),
            pl.BlockSpec((D, KVH * HD), lambda i: (0, 0)),
            pl.BlockSpec((D, KVH * HD), lambda i: (0, 0)),
            pl.BlockSpec((SB, H * HD), lambda i: (i, 0)),
            pl.BlockSpec((SB, H * HD), lambda i: (i, 0)),
            pl.BlockSpec((SB, KVH * HD), lambda i: (i, 0)),
            pl.BlockSpec((SB, KVH * HD), lambda i: (i, 0)),
        ],
        out_specs=[
            pl.BlockSpec((SB, H * HD), lambda i: (i, 0)),
            pl.BlockSpec((SB, KVH * HD), lambda i: (i, 0)),
            pl.BlockSpec((SB, KVH * HD), lambda i: (i, 0)),
        ],
        out_shape=[
            jax.ShapeDtypeStruct((S, H * HD), jnp.bfloat16),
            jax.ShapeDtypeStruct((S, KVH * HD), jnp.bfloat16),
            jax.ShapeDtypeStruct((S, KVH * HD), jnp.bfloat16),
        ],
    )(x, ln1_w, wq, wk, wv, cos_q, sin_q, cos_k, sin_k)


def _attn_kernel(q_ref, k_ref, v_ref, o_ref, acc_ref, m_ref, l_ref):
    qb = pl.program_id(1)
    kb = pl.program_id(2)

    @pl.when(kb == 0)
    def _init():
        acc_ref[...] = jnp.zeros_like(acc_ref)
        m_ref[...] = jnp.full_like(m_ref, NEG)
        l_ref[...] = jnp.zeros_like(l_ref)

    @pl.when(kb <= qb)
    def _compute():
        q = q_ref[0]
        k = k_ref[0]
        s = jax.lax.dot_general(q, k, (((1,), (1,)), ((), ())),
                                preferred_element_type=jnp.float32)
        s = s * (1.0 / math.sqrt(HD))
        row = qb * BQ + jax.lax.broadcasted_iota(jnp.int32, (BQ, BK), 0)
        col = kb * BK + jax.lax.broadcasted_iota(jnp.int32, (BQ, BK), 1)
        s = jnp.where(row >= col, s, NEG)
        m_prev = m_ref[...]
        m_cur = jnp.max(s, axis=-1, keepdims=True)
        m_new = jnp.maximum(m_prev, m_cur)
        p = jnp.exp(s - m_new)
        alpha = jnp.exp(m_prev - m_new)
        l_ref[...] = l_ref[...] * alpha + jnp.sum(p, axis=-1, keepdims=True)
        acc_ref[...] = acc_ref[...] * alpha + jnp.dot(
            p.astype(jnp.bfloat16), v_ref[0],
            preferred_element_type=jnp.float32)
        m_ref[...] = m_new

    @pl.when(kb == pl.num_programs(2) - 1)
    def _final():
        o_ref[0] = (acc_ref[...] / l_ref[...]).astype(jnp.bfloat16)


def _attention(q, k, v):
    return pl.pallas_call(
        _attn_kernel,
        grid=(H, S // BQ, S // BK),
        in_specs=[
            pl.BlockSpec((1, BQ, HD), lambda h, i, j: (h, i, 0)),
            pl.BlockSpec((1, BK, HD), lambda h, i, j: (h // REP, j, 0)),
            pl.BlockSpec((1, BK, HD), lambda h, i, j: (h // REP, j, 0)),
        ],
        out_specs=pl.BlockSpec((1, BQ, HD), lambda h, i, j: (h, i, 0)),
        out_shape=jax.ShapeDtypeStruct((H, S, HD), jnp.bfloat16),
        scratch_shapes=[
            pltpu.VMEM((BQ, HD), jnp.float32),
            pltpu.VMEM((BQ, 1), jnp.float32),
            pltpu.VMEM((BQ, 1), jnp.float32),
        ],
    )(q, k, v)


def _post_kernel(x_ref, attn_ref, wo_ref, ln2_ref, gate_ref,
                 hid_ref, x2_ref, wgt_ref):
    ao = jnp.dot(attn_ref[...], wo_ref[...],
                 preferred_element_type=jnp.float32)
    hid = x_ref[...] + ao
    hid_ref[...] = hid
    x2 = _rms(hid, ln2_ref[...])
    x2_ref[...] = x2.astype(jnp.bfloat16)
    logits = jnp.dot(x2, gate_ref[...], preferred_element_type=jnp.float32)
    mx = jnp.max(logits, axis=-1, keepdims=True)
    p = jnp.exp(logits - mx)
    p = p / jnp.sum(p, axis=-1, keepdims=True)
    lane = jax.lax.broadcasted_iota(jnp.int32, p.shape, 1)
    m1 = jnp.max(p, axis=-1, keepdims=True)
    e1 = jnp.min(jnp.where(p == m1, lane, E), axis=-1, keepdims=True)
    p2 = jnp.where(lane == e1, -1.0, p)
    m2 = jnp.max(p2, axis=-1, keepdims=True)
    e2 = jnp.min(jnp.where(p2 == m2, lane, E), axis=-1, keepdims=True)
    denom = m1 + m2
    wgt = (jnp.where(lane == e1, m1, 0.0)
           + jnp.where(lane == e2, m2, 0.0)) / denom
    wgt_ref[...] = wgt


def _post_attn(x, attn, wo, ln2_w, gate_w):
    return pl.pallas_call(
        _post_kernel,
        grid=(NSB,),
        in_specs=[
            pl.BlockSpec((SB, D), lambda i: (i, 0)),
            pl.BlockSpec((SB, H * HD), lambda i: (i, 0)),
            pl.BlockSpec((H * HD, D), lambda i: (0, 0)),
            pl.BlockSpec((1, D), lambda i: (0, 0)),
            pl.BlockSpec((D, E), lambda i: (0, 0)),
        ],
        out_specs=[
            pl.BlockSpec((SB, D), lambda i: (i, 0)),
            pl.BlockSpec((SB, D), lambda i: (i, 0)),
            pl.BlockSpec((SB, E), lambda i: (i, 0)),
        ],
        out_shape=[
            jax.ShapeDtypeStruct((S, D), jnp.float32),
            jax.ShapeDtypeStruct((S, D), jnp.bfloat16),
            jax.ShapeDtypeStruct((S, E), jnp.float32),
        ],
    )(x, attn, wo, ln2_w, gate_w)


def _moe_kernel(x2_ref, w1_ref, w3_ref, w2_ref, wgt_ref, hid_ref, out_ref):
    e = pl.program_id(0)
    x2 = x2_ref[...]
    a = jnp.dot(x2, w1_ref[0], preferred_element_type=jnp.float32)
    b = jnp.dot(x2, w3_ref[0], preferred_element_type=jnp.float32)
    g = (a * jax.lax.logistic(a)) * b
    eo = jnp.dot(g.astype(jnp.bfloat16), w2_ref[0],
                 preferred_element_type=jnp.float32)
    lane = jax.lax.broadcasted_iota(jnp.int32, (SB, E), 1)
    wcol = jnp.sum(jnp.where(lane == e, wgt_ref[...], 0.0),
                   axis=-1, keepdims=True)
    contrib = wcol * eo

    @pl.when(e == 0)
    def _init():
        out_ref[...] = hid_ref[...] + contrib

    @pl.when(e != 0)
    def _acc():
        out_ref[...] = out_ref[...] + contrib


def _moe(x2, w1, w3, w2, wgt, hidden):
    return pl.pallas_call(
        _moe_kernel,
        grid=(E, NSB),
        in_specs=[
            pl.BlockSpec((SB, D), lambda e, i: (i, 0)),
            pl.BlockSpec((1, D, F), lambda e, i: (e, 0, 0)),
            pl.BlockSpec((1, D, F), lambda e, i: (e, 0, 0)),
            pl.BlockSpec((1, F, D), lambda e, i: (e, 0, 0)),
            pl.BlockSpec((SB, E), lambda e, i: (i, 0)),
            pl.BlockSpec((SB, D), lambda e, i: (i, 0)),
        ],
        out_specs=pl.BlockSpec((SB, D), lambda e, i: (i, 0)),
        out_shape=jax.ShapeDtypeStruct((S, D), jnp.float32),
    )(x2, w1, w3, w2, wgt, hidden)


def kernel(hidden_states, attention_mask, position_ids, ln1_w, ln2_w,
           wq, wk, wv, wo, gate_w, w1, w3, w2):
    x = hidden_states.reshape(S, D)

    inv_freq = 1.0 / (THETA ** (jnp.arange(0, HD, 2, dtype=jnp.float32) / HD))
    pos = position_ids.reshape(S).astype(jnp.float32)
    freqs = pos[:, None] * inv_freq[None, :]
    emb = jnp.concatenate([freqs, freqs], axis=-1)
    cos, sin = jnp.cos(emb), jnp.sin(emb)
    cos_q = jnp.tile(cos, (1, H))
    sin_q = jnp.tile(sin, (1, H))
    cos_k = jnp.tile(cos, (1, KVH))
    sin_k = jnp.tile(sin, (1, KVH))

    wq_b = wq.astype(jnp.bfloat16)
    wk_b = wk.astype(jnp.bfloat16)
    wv_b = wv.astype(jnp.bfloat16)
    wo_b = wo.astype(jnp.bfloat16)
    w1_b = w1.astype(jnp.bfloat16)
    w3_b = w3.astype(jnp.bfloat16)
    w2_b = w2.astype(jnp.bfloat16)

    q, k, v = _qkv(x, ln1_w.reshape(1, D), wq_b, wk_b, wv_b,
                   cos_q, sin_q, cos_k, sin_k)

    qh = q.reshape(S, H, HD).transpose(1, 0, 2)
    kh = k.reshape(S, KVH, HD).transpose(1, 0, 2)
    vh = v.reshape(S, KVH, HD).transpose(1, 0, 2)
    attn = _attention(qh, kh, vh)
    attn2 = attn.transpose(1, 0, 2).reshape(S, H * HD)

    hidden, x2, wgt = _post_attn(x, attn2, wo_b, ln2_w.reshape(1, D), gate_w)

    out = _moe(x2, w1_b, w3_b, w2_b, wgt, hidden)
    return out.reshape(B, S, D)


# speedup vs baseline: 1.1103x; 1.1103x over previous
import functools
import math

import jax
import jax.numpy as jnp
import numpy as np
from jax.experimental import pallas as pl
from jax.experimental.pallas import tpu as pltpu

B, S, D = 1, 2048, 768
H, KVH, HD = 12, 4, 64
E, TOPK, F = 8, 2, 2048
EPS, THETA = 1e-6, 10000.0
REP = H // KVH

SB = 512
NSB = S // SB
BQ = 512
BK = 512
NEG = -1e30


def _rms(x, w):
    var = jnp.mean(jnp.square(x), axis=-1, keepdims=True)
    return (x * jax.lax.rsqrt(var + EPS)) * w


def _rot_half_heads(x):
    half = HD // 2
    s = jnp.concatenate([x[:, half:], x[:, :half]], axis=1)
    t = jnp.concatenate([x[:, -half:], x[:, :-half]], axis=1)
    lane = jax.lax.broadcasted_iota(jnp.int32, x.shape, 1)
    first = (lane % HD) < half
    return jnp.where(first, -s, t)


def _qkv_kernel(x_ref, ln1_ref, wq_ref, wk_ref, wv_ref, cq_ref, sq_ref,
                ck_ref, sk_ref, q_ref, k_ref, v_ref):
    h = _rms(x_ref[...], ln1_ref[...])
    hb = h.astype(jnp.bfloat16)
    q = jnp.dot(hb, wq_ref[...], preferred_element_type=jnp.float32)
    k = jnp.dot(hb, wk_ref[...], preferred_element_type=jnp.float32)
    v = jnp.dot(hb, wv_ref[...], preferred_element_type=jnp.float32)
    q = q * cq_ref[...] + _rot_half_heads(q) * sq_ref[...]
    k = k * ck_ref[...] + _rot_half_heads(k) * sk_ref[...]
    q_ref[...] = q.astype(jnp.bfloat16)
    k_ref[...] = k.astype(jnp.bfloat16)
    v_ref[...] = v.astype(jnp.bfloat16)


def _qkv(x, ln1_w, wq, wk, wv, cos_q, sin_q, cos_k, sin_k):
    return pl.pallas_call(
        _qkv_kernel,
        grid=(NSB,),
        in_specs=[
            pl.BlockSpec((SB, D), lambda i: (i, 0)),
            pl.BlockSpec((1, D), lambda i: (0, 0)),
            pl.BlockSpec((D, H * HD), lambda i: (0, 0)),
            pl.BlockSpec((D, KVH * HD), lambda i: (0, 0)),
            pl.BlockSpec((D, KVH * HD), lambda i: (0, 0)),
            pl.BlockSpec((SB, H * HD), lambda i: (i, 0)),
            pl.BlockSpec((SB, H * HD), lambda i: (i, 0)),
            pl.BlockSpec((SB, KVH * HD), lambda i: (i, 0)),
            pl.BlockSpec((SB, KVH * HD), lambda i: (i, 0)),
        ],
        out_specs=[
            pl.BlockSpec((SB, H * HD), lambda i: (i, 0)),
            pl.BlockSpec((SB, KVH * HD), lambda i: (i, 0)),
            pl.BlockSpec((SB, KVH * HD), lambda i: (i, 0)),
        ],
        out_shape=[
            jax.ShapeDtypeStruct((S, H * HD), jnp.bfloat16),
            jax.ShapeDtypeStruct((S, KVH * HD), jnp.bfloat16),
            jax.ShapeDtypeStruct((S, KVH * HD), jnp.bfloat16),
        ],
    )(x, ln1_w, wq, wk, wv, cos_q, sin_q, cos_k, sin_k)


def _attn_kernel(q_ref, k_ref, v_ref, o_ref, acc_ref, m_ref, l_ref):
    qb = pl.program_id(1)
    kb = pl.program_id(2)

    @pl.when(kb == 0)
    def _init():
        acc_ref[...] = jnp.zeros_like(acc_ref)
        m_ref[...] = jnp.full_like(m_ref, NEG)
        l_ref[...] = jnp.zeros_like(l_ref)

    @pl.when(kb <= qb)
    def _compute():
        q = q_ref[0]
        k = k_ref[0]
        s = jax.lax.dot_general(q, k, (((1,), (1,)), ((), ())),
                                preferred_element_type=jnp.float32)
        s = s * (1.0 / math.sqrt(HD))
        row = qb * BQ + jax.lax.broadcasted_iota(jnp.int32, (BQ, BK), 0)
        col = kb * BK + jax.lax.broadcasted_iota(jnp.int32, (BQ, BK), 1)
        s = jnp.where(row >= col, s, NEG)
        m_prev = m_ref[...]
        m_cur = jnp.max(s, axis=-1, keepdims=True)
        m_new = jnp.maximum(m_prev, m_cur)
        p = jnp.exp(s - m_new)
        alpha = jnp.exp(m_prev - m_new)
        l_ref[...] = l_ref[...] * alpha + jnp.sum(p, axis=-1, keepdims=True)
        acc_ref[...] = acc_ref[...] * alpha + jnp.dot(
            p.astype(jnp.bfloat16), v_ref[0],
            preferred_element_type=jnp.float32)
        m_ref[...] = m_new

    @pl.when(kb == pl.num_programs(2) - 1)
    def _final():
        o_ref[0] = (acc_ref[...] / l_ref[...]).astype(jnp.bfloat16)


def _attention(q, k, v):
    return pl.pallas_call(
        _attn_kernel,
        grid=(H, S // BQ, S // BK),
        in_specs=[
            pl.BlockSpec((1, BQ, HD), lambda h, i, j: (h, i, 0)),
            pl.BlockSpec((1, BK, HD), lambda h, i, j: (h // REP, j, 0)),
            pl.BlockSpec((1, BK, HD), lambda h, i, j: (h // REP, j, 0)),
        ],
        out_specs=pl.BlockSpec((1, BQ, HD), lambda h, i, j: (h, i, 0)),
        out_shape=jax.ShapeDtypeStruct((H, S, HD), jnp.bfloat16),
        scratch_shapes=[
            pltpu.VMEM((BQ, HD), jnp.float32),
            pltpu.VMEM((BQ, 1), jnp.float32),
            pltpu.VMEM((BQ, 1), jnp.float32),
        ],
    )(q, k, v)


def _post_kernel(x_ref, attn_ref, wo_ref, ln2_ref, gate_ref,
                 hid_ref, x2_ref, wgt_ref):
    ao = jnp.dot(attn_ref[...], wo_ref[...],
                 preferred_element_type=jnp.float32)
    hid = x_ref[...] + ao
    hid_ref[...] = hid
    x2 = _rms(hid, ln2_ref[...])
    x2_ref[...] = x2.astype(jnp.bfloat16)
    logits = jnp.dot(x2, gate_ref[...], preferred_element_type=jnp.float32)
    mx = jnp.max(logits, axis=-1, keepdims=True)
    p = jnp.exp(logits - mx)
    p = p / jnp.sum(p, axis=-1, keepdims=True)
    lane = jax.lax.broadcasted_iota(jnp.int32, p.shape, 1)
    m1 = jnp.max(p, axis=-1, keepdims=True)
    e1 = jnp.min(jnp.where(p == m1, lane, E), axis=-1, keepdims=True)
    p2 = jnp.where(lane == e1, -1.0, p)
    m2 = jnp.max(p2, axis=-1, keepdims=True)
    e2 = jnp.min(jnp.where(p2 == m2, lane, E), axis=-1, keepdims=True)
    denom = m1 + m2
    wgt = (jnp.where(lane == e1, m1, 0.0)
           + jnp.where(lane == e2, m2, 0.0)) / denom
    wgt_ref[...] = wgt


def _post_attn(x, attn, wo, ln2_w, gate_w):
    return pl.pallas_call(
        _post_kernel,
        grid=(NSB,),
        in_specs=[
            pl.BlockSpec((SB, D), lambda i: (i, 0)),
            pl.BlockSpec((SB, H * HD), lambda i: (i, 0)),
            pl.BlockSpec((H * HD, D), lambda i: (0, 0)),
            pl.BlockSpec((1, D), lambda i: (0, 0)),
            pl.BlockSpec((D, E), lambda i: (0, 0)),
        ],
        out_specs=[
            pl.BlockSpec((SB, D), lambda i: (i, 0)),
            pl.BlockSpec((SB, D), lambda i: (i, 0)),
            pl.BlockSpec((SB, E), lambda i: (i, 0)),
        ],
        out_shape=[
            jax.ShapeDtypeStruct((S, D), jnp.float32),
            jax.ShapeDtypeStruct((S, D), jnp.bfloat16),
            jax.ShapeDtypeStruct((S, E), jnp.float32),
        ],
    )(x, attn, wo, ln2_w, gate_w)


def _moe_kernel(x2_ref, w1_ref, w3_ref, w2_ref, wgt_ref, hid_ref, out_ref):
    e = pl.program_id(1)
    x2 = x2_ref[...]
    a = jnp.dot(x2, w1_ref[0], preferred_element_type=jnp.float32)
    b = jnp.dot(x2, w3_ref[0], preferred_element_type=jnp.float32)
    g = (a * jax.lax.logistic(a)) * b
    eo = jnp.dot(g.astype(jnp.bfloat16), w2_ref[0],
                 preferred_element_type=jnp.float32)
    lane = jax.lax.broadcasted_iota(jnp.int32, (SB, E), 1)
    wcol = jnp.sum(jnp.where(lane == e, wgt_ref[...], 0.0),
                   axis=-1, keepdims=True)
    contrib = wcol * eo

    @pl.when(e == 0)
    def _init():
        out_ref[...] = hid_ref[...] + contrib

    @pl.when(e != 0)
    def _acc():
        out_ref[...] = out_ref[...] + contrib


def _moe(x2, w1, w3, w2, wgt, hidden):
    return pl.pallas_call(
        _moe_kernel,
        grid=(NSB, E),
        in_specs=[
            pl.BlockSpec((SB, D), lambda i, e: (i, 0)),
            pl.BlockSpec((1, D, F), lambda i, e: (e, 0, 0)),
            pl.BlockSpec((1, D, F), lambda i, e: (e, 0, 0)),
            pl.BlockSpec((1, F, D), lambda i, e: (e, 0, 0)),
            pl.BlockSpec((SB, E), lambda i, e: (i, 0)),
            pl.BlockSpec((SB, D), lambda i, e: (i, 0)),
        ],
        out_specs=pl.BlockSpec((SB, D), lambda i, e: (i, 0)),
        out_shape=jax.ShapeDtypeStruct((S, D), jnp.float32),
    )(x2, w1, w3, w2, wgt, hidden)


def kernel(hidden_states, attention_mask, position_ids, ln1_w, ln2_w,
           wq, wk, wv, wo, gate_w, w1, w3, w2):
    x = hidden_states.reshape(S, D)

    inv_freq = 1.0 / (THETA ** (jnp.arange(0, HD, 2, dtype=jnp.float32) / HD))
    pos = position_ids.reshape(S).astype(jnp.float32)
    freqs = pos[:, None] * inv_freq[None, :]
    emb = jnp.concatenate([freqs, freqs], axis=-1)
    cos, sin = jnp.cos(emb), jnp.sin(emb)
    cos_q = jnp.tile(cos, (1, H))
    sin_q = jnp.tile(sin, (1, H))
    cos_k = jnp.tile(cos, (1, KVH))
    sin_k = jnp.tile(sin, (1, KVH))

    wq_b = wq.astype(jnp.bfloat16)
    wk_b = wk.astype(jnp.bfloat16)
    wv_b = wv.astype(jnp.bfloat16)
    wo_b = wo.astype(jnp.bfloat16)
    w1_b = w1.astype(jnp.bfloat16)
    w3_b = w3.astype(jnp.bfloat16)
    w2_b = w2.astype(jnp.bfloat16)

    q, k, v = _qkv(x, ln1_w.reshape(1, D), wq_b, wk_b, wv_b,
                   cos_q, sin_q, cos_k, sin_k)

    qh = q.reshape(S, H, HD).transpose(1, 0, 2)
    kh = k.reshape(S, KVH, HD).transpose(1, 0, 2)
    vh = v.reshape(S, KVH, HD).transpose(1, 0, 2)
    attn = _attention(qh, kh, vh)
    attn2 = attn.transpose(1, 0, 2).reshape(S, H * HD)

    hidden, x2, wgt = _post_attn(x, attn2, wo_b, ln2_w.reshape(1, D), gate_w)

    out = _moe(x2, w1_b, w3_b, w2_b, wgt, hidden)
    return out.reshape(B, S, D)


# speedup vs baseline: 1.1818x; 1.0644x over previous
import functools
import math

import jax
import jax.numpy as jnp
import numpy as np
from jax import lax
from jax.experimental import pallas as pl
from jax.experimental.pallas import tpu as pltpu
from jax.experimental.pallas import tpu_sc as plsc

B, S, D = 1, 2048, 768
H, KVH, HD = 12, 4, 64
E, TOPK, F = 8, 2, 2048
EPS, THETA = 1e-6, 10000.0
REP = H // KVH

SB = 512
NSB = S // SB
BQ = 512
BK = 512
NEG = -1e30

NW = 32
TPW = S // NW
PPW = 2 * TPW
NP = 2 * S
BLK = 128
NB = NP // BLK + E
PP = NB * BLK
NBP = 48
LANES = 16


def _rms(x, w):
    var = jnp.mean(jnp.square(x), axis=-1, keepdims=True)
    return (x * jax.lax.rsqrt(var + EPS)) * w


def _rot_half_heads(x):
    half = HD // 2
    s = jnp.concatenate([x[:, half:], x[:, :half]], axis=1)
    t = jnp.concatenate([x[:, -half:], x[:, :-half]], axis=1)
    lane = jax.lax.broadcasted_iota(jnp.int32, x.shape, 1)
    first = (lane % HD) < half
    return jnp.where(first, -s, t)


def _qkv_kernel(x_ref, ln1_ref, wq_ref, wk_ref, wv_ref, cq_ref, sq_ref,
                ck_ref, sk_ref, q_ref, k_ref, v_ref):
    h = _rms(x_ref[...], ln1_ref[...])
    hb = h.astype(jnp.bfloat16)
    q = jnp.dot(hb, wq_ref[...], preferred_element_type=jnp.float32)
    k = jnp.dot(hb, wk_ref[...], preferred_element_type=jnp.float32)
    v = jnp.dot(hb, wv_ref[...], preferred_element_type=jnp.float32)
    q = q * cq_ref[...] + _rot_half_heads(q) * sq_ref[...]
    k = k * ck_ref[...] + _rot_half_heads(k) * sk_ref[...]
    q_ref[...] = q.astype(jnp.bfloat16)
    k_ref[...] = k.astype(jnp.bfloat16)
    v_ref[...] = v.astype(jnp.bfloat16)


def _qkv(x, ln1_w, wq, wk, wv, cos_q, sin_q, cos_k, sin_k):
    return pl.pallas_call(
        _qkv_kernel,
        grid=(NSB,),
        in_specs=[
            pl.BlockSpec((SB, D), lambda i: (i, 0)),
            pl.BlockSpec((1, D), lambda i: (0, 0)),
            pl.BlockSpec((D, H * HD), lambda i: (0, 0)),
            pl.BlockSpec((D, KVH * HD), lambda i: (0, 0)),
            pl.BlockSpec((D, KVH * HD), lambda i: (0, 0)),
            pl.BlockSpec((SB, H * HD), lambda i: (i, 0)),
            pl.BlockSpec((SB, H * HD), lambda i: (i, 0)),
            pl.BlockSpec((SB, KVH * HD), lambda i: (i, 0)),
            pl.BlockSpec((SB, KVH * HD), lambda i: (i, 0)),
        ],
        out_specs=[
            pl.BlockSpec((SB, H * HD), lambda i: (i, 0)),
            pl.BlockSpec((SB, KVH * HD), lambda i: (i, 0)),
            pl.BlockSpec((SB, KVH * HD), lambda i: (i, 0)),
        ],
        out_shape=[
            jax.ShapeDtypeStruct((S, H * HD), jnp.bfloat16),
            jax.ShapeDtypeStruct((S, KVH * HD), jnp.bfloat16),
            jax.ShapeDtypeStruct((S, KVH * HD), jnp.bfloat16),
        ],
    )(x, ln1_w, wq, wk, wv, cos_q, sin_q, cos_k, sin_k)


def _attn_kernel(q_ref, k_ref, v_ref, o_ref, acc_ref, m_ref, l_ref):
    qb = pl.program_id(1)
    kb = pl.program_id(2)

    @pl.when(kb == 0)
    def _init():
        acc_ref[...] = jnp.zeros_like(acc_ref)
        m_ref[...] = jnp.full_like(m_ref, NEG)
        l_ref[...] = jnp.zeros_like(l_ref)

    @pl.when(kb <= qb)
    def _compute():
        q = q_ref[0]
        k = k_ref[0]
        s = jax.lax.dot_general(q, k, (((1,), (1,)), ((), ())),
                                preferred_element_type=jnp.float32)
        s = s * (1.0 / math.sqrt(HD))
        row = qb * BQ + jax.lax.broadcasted_iota(jnp.int32, (BQ, BK), 0)
        col = kb * BK + jax.lax.broadcasted_iota(jnp.int32, (BQ, BK), 1)
        s = jnp.where(row >= col, s, NEG)
        m_prev = m_ref[...]
        m_cur = jnp.max(s, axis=-1, keepdims=True)
        m_new = jnp.maximum(m_prev, m_cur)
        p = jnp.exp(s - m_new)
        alpha = jnp.exp(m_prev - m_new)
        l_ref[...] = l_ref[...] * alpha + jnp.sum(p, axis=-1, keepdims=True)
        acc_ref[...] = acc_ref[...] * alpha + jnp.dot(
            p.astype(jnp.bfloat16), v_ref[0],
            preferred_element_type=jnp.float32)
        m_ref[...] = m_new

    @pl.when(kb == pl.num_programs(2) - 1)
    def _final():
        o_ref[0] = (acc_ref[...] / l_ref[...]).astype(jnp.bfloat16)


def _attention(q, k, v):
    return pl.pallas_call(
        _attn_kernel,
        grid=(H, S // BQ, S // BK),
        in_specs=[
            pl.BlockSpec((1, BQ, HD), lambda h, i, j: (h, i, 0)),
            pl.BlockSpec((1, BK, HD), lambda h, i, j: (h // REP, j, 0)),
            pl.BlockSpec((1, BK, HD), lambda h, i, j: (h // REP, j, 0)),
        ],
        out_specs=pl.BlockSpec((1, BQ, HD), lambda h, i, j: (h, i, 0)),
        out_shape=jax.ShapeDtypeStruct((H, S, HD), jnp.bfloat16),
        scratch_shapes=[
            pltpu.VMEM((BQ, HD), jnp.float32),
            pltpu.VMEM((BQ, 1), jnp.float32),
            pltpu.VMEM((BQ, 1), jnp.float32),
        ],
    )(q, k, v)


def _post_kernel(x_ref, attn_ref, wo_ref, ln2_ref, gate_ref,
                 hid_ref, x2_ref, sel_ref, rw_ref, cnt_ref):
    ao = jnp.dot(attn_ref[...], wo_ref[...],
                 preferred_element_type=jnp.float32)
    hid = x_ref[...] + ao
    hid_ref[...] = hid
    x2 = _rms(hid, ln2_ref[...])
    x2_ref[...] = x2
    logits = jax.lax.dot_general(
        x2, gate_ref[...], (((1,), (0,)), ((), ())),
        preferred_element_type=jnp.float32,
        precision=jax.lax.Precision.HIGHEST)
    mx = jnp.max(logits, axis=-1, keepdims=True)
    p = jnp.exp(logits - mx)
    p = p / jnp.sum(p, axis=-1, keepdims=True)
    lane = jax.lax.broadcasted_iota(jnp.int32, p.shape, 1)
    m1 = jnp.max(p, axis=-1, keepdims=True)
    e1 = jnp.min(jnp.where(p == m1, lane, E), axis=-1, keepdims=True)
    p2 = jnp.where(lane == e1, -1.0, p)
    m2 = jnp.max(p2, axis=-1, keepdims=True)
    e2 = jnp.min(jnp.where(p2 == m2, lane, E), axis=-1, keepdims=True)
    denom = m1 + m2
    sel_ref[...] = jnp.concatenate([e1, e2], axis=1)
    rw_ref[...] = jnp.concatenate([m1 / denom, m2 / denom], axis=1)
    lane16 = jax.lax.broadcasted_iota(jnp.int32, (SB, LANES), 1)
    oh = ((lane16 == e1).astype(jnp.bfloat16)
          + (lane16 == e2).astype(jnp.bfloat16))
    nch = SB // TPW
    ar = jax.lax.broadcasted_iota(jnp.int32, (nch, SB), 0)
    ac = jax.lax.broadcasted_iota(jnp.int32, (nch, SB), 1) // TPW
    seg = (ar == ac).astype(jnp.bfloat16)
    cnt_ref[...] = jnp.dot(seg, oh,
                           preferred_element_type=jnp.float32).astype(jnp.int32)


def _post_attn(x, attn, wo, ln2_w, gate_w):
    nch = SB // TPW
    return pl.pallas_call(
        _post_kernel,
        grid=(NSB,),
        in_specs=[
            pl.BlockSpec((SB, D), lambda i: (i, 0)),
            pl.BlockSpec((SB, H * HD), lambda i: (i, 0)),
            pl.BlockSpec((H * HD, D), lambda i: (0, 0)),
            pl.BlockSpec((1, D), lambda i: (0, 0)),
            pl.BlockSpec((D, E), lambda i: (0, 0)),
        ],
        out_specs=[
            pl.BlockSpec((SB, D), lambda i: (i, 0)),
            pl.BlockSpec((SB, D), lambda i: (i, 0)),
            pl.BlockSpec((SB, TOPK), lambda i: (i, 0)),
            pl.BlockSpec((SB, TOPK), lambda i: (i, 0)),
            pl.BlockSpec((nch, LANES), lambda i: (i, 0)),
        ],
        out_shape=[
            jax.ShapeDtypeStruct((S, D), jnp.float32),
            jax.ShapeDtypeStruct((S, D), jnp.float32),
            jax.ShapeDtypeStruct((S, TOPK), jnp.int32),
            jax.ShapeDtypeStruct((S, TOPK), jnp.float32),
            jax.ShapeDtypeStruct((NW, LANES), jnp.int32),
        ],
    )(x, attn, wo, ln2_w, gate_w)


def _bc(vec, lane):
    idx = jnp.zeros((LANES,), jnp.int32) + lane
    return vec.at[idx].get(mode="promise_in_bounds")


def _dispatch_body(sel_hbm, cnt_hbm, x2_hbm,
                   xs_hbm, inv_hbm, bexp_hbm, nrows_hbm,
                   selv, k0v, k1v, cntv, t1v, t2v, x2rows, sem):
    nc = 2
    wid = lax.axis_index("s") * nc + lax.axis_index("c")
    iota = jnp.arange(LANES, dtype=jnp.int32)
    zero = jnp.zeros((LANES,), jnp.int32)
    widv = zero + wid

    pltpu.sync_copy(cnt_hbm, cntv)
    tot = jnp.zeros((LANES,), jnp.int32)
    pfx = jnp.zeros((LANES,), jnp.int32)
    for ch in range(NW):
        row = cntv[ch]
        tot = tot + row
        pfx = pfx + jnp.where((zero + ch) < widv, row, 0)
    nb = (tot + (BLK - 1)) // BLK
    blk_end = plsc.cumsum(nb)
    blk_start = blk_end - nb
    roff = blk_start * BLK + pfx

    pltpu.sync_copy(sel_hbm.at[pl.ds(wid * TPW, TPW)],
                    selv.at[pl.ds(0, TPW)])
    pltpu.sync_copy(sel_hbm.at[pl.ds(S + wid * TPW, TPW)],
                    selv.at[pl.ds(TPW, TPW)])
    offs = [_bc(roff, e) for e in range(E)]
    runs = [jnp.zeros((LANES,), jnp.int32) for _ in range(E)]
    nchunk = PPW // LANES
    for c in range(nchunk):
        v = selv[pl.ds(c * LANES, LANES)]
        dest = jnp.zeros((LANES,), jnp.int32)
        for e in range(E):
            m = v == e
            csum = plsc.cumsum(jnp.where(m, 1, 0))
            dest = jnp.where(m, offs[e] + runs[e] + csum - 1, dest)
            runs[e] = runs[e] + plsc.all_reduce_population_count(m)
        if c < nchunk // 2:
            k0v[pl.ds(c * LANES, LANES)] = dest
        else:
            k1v[pl.ds((c - nchunk // 2) * LANES, LANES)] = dest
    pltpu.sync_copy(k0v, inv_hbm.at[pl.ds(wid * TPW, TPW)])
    pltpu.sync_copy(k1v, inv_hbm.at[pl.ds(S + wid * TPW, TPW)])

    pltpu.sync_copy(x2_hbm.at[pl.ds(wid * TPW, TPW)], x2rows)
    pltpu.async_copy(x2rows, xs_hbm.at[k0v], sem).wait()
    pltpu.async_copy(x2rows, xs_hbm.at[k1v], sem).wait()

    @pl.when(wid < NBP // LANES)
    def _tables():
        b_vec = zero + wid * LANES + iota
        be = jnp.zeros((LANES,), jnp.int32)
        for e in range(E):
            be = be + jnp.where(b_vec >= _bc(blk_end, e), 1, 0)
        t1v[...] = jnp.minimum(be, E - 1)
        nr = jnp.zeros((LANES,), jnp.int32)
        for e in range(E):
            rem = (_bc(tot, e) - (b_vec - _bc(blk_start, e)) * BLK)
            nr = jnp.where(be == e, jnp.clip(rem, 0, BLK), nr)
        t2v[...] = nr
        pltpu.sync_copy(t1v, bexp_hbm.at[pl.ds(wid * LANES, LANES)])
        pltpu.sync_copy(t2v, nrows_hbm.at[pl.ds(wid * LANES, LANES)])


def _dispatch(sel_flat, cnt, x2f):
    mesh = plsc.VectorSubcoreMesh(core_axis_name="c", subcore_axis_name="s")
    f = pl.kernel(
        _dispatch_body,
        out_type=[
            jax.ShapeDtypeStruct((PP, D), jnp.float32),
            jax.ShapeDtypeStruct((NP,), jnp.int32),
            jax.ShapeDtypeStruct((NBP,), jnp.int32),
            jax.ShapeDtypeStruct((NBP,), jnp.int32),
        ],
        mesh=mesh,
        compiler_params=pltpu.CompilerParams(needs_layout_passes=False),
        scratch_types=[
            pltpu.VMEM((PPW,), jnp.int32),
            pltpu.VMEM((TPW,), jnp.int32),
            pltpu.VMEM((TPW,), jnp.int32),
            pltpu.VMEM((NW, LANES), jnp.int32),
            pltpu.VMEM((LANES,), jnp.int32),
            pltpu.VMEM((LANES,), jnp.int32),
            pltpu.VMEM((TPW, D), jnp.float32),
            pltpu.SemaphoreType.DMA,
        ],
    )
    return f(sel_flat, cnt, x2f)


def _gmm_kernel(bexp_ref, nrows_ref, xs_ref, w1_ref, w3_ref, w2_ref, ys_ref):
    b = pl.program_id(0)
    n = nrows_ref[b]

    @pl.when(n > 0)
    def _compute():
        rowi = jax.lax.broadcasted_iota(jnp.int32, (BLK, D), 0)
        x = jnp.where(rowi < n, xs_ref[...], 0.0).astype(jnp.bfloat16)
        a = jnp.dot(x, w1_ref[0], preferred_element_type=jnp.float32)
        t = jnp.dot(x, w3_ref[0], preferred_element_type=jnp.float32)
        g = (a * jax.lax.logistic(a)) * t
        ys_ref[...] = jnp.dot(g.astype(jnp.bfloat16), w2_ref[0],
                              preferred_element_type=jnp.float32)


def _gmm(bexp, nrows, xs, w1, w3, w2):
    grid_spec = pltpu.PrefetchScalarGridSpec(
        num_scalar_prefetch=2,
        grid=(NB,),
        in_specs=[
            pl.BlockSpec((BLK, D), lambda b, be, nr: (b, 0)),
            pl.BlockSpec((1, D, F), lambda b, be, nr: (be[b], 0, 0)),
            pl.BlockSpec((1, D, F), lambda b, be, nr: (be[b], 0, 0)),
            pl.BlockSpec((1, F, D), lambda b, be, nr: (be[b], 0, 0)),
        ],
        out_specs=pl.BlockSpec((BLK, D), lambda b, be, nr: (b, 0)),
    )
    return pl.pallas_call(
        _gmm_kernel,
        grid_spec=grid_spec,
        out_shape=jax.ShapeDtypeStruct((PP, D), jnp.float32),
    )(bexp, nrows, xs, w1, w3, w2)


def _combine_body(hid_hbm, ys_hbm, inv_hbm, rw_hbm, out_hbm,
                  i0v, i1v, rw0v, rw1v, rows0, rows1, hidv, outv, sem):
    nc = 2
    wid = lax.axis_index("s") * nc + lax.axis_index("c")
    zero = jnp.zeros((LANES,), jnp.int32)
    half = TPW // 2
    for h in range(2):
        tbase = wid * TPW + h * half
        pltpu.sync_copy(inv_hbm.at[pl.ds(tbase, half)], i0v)
        pltpu.sync_copy(inv_hbm.at[pl.ds(S + tbase, half)], i1v)
        pltpu.async_copy(ys_hbm.at[i0v], rows0, sem).wait()
        pltpu.async_copy(ys_hbm.at[i1v], rows1, sem).wait()
        pltpu.sync_copy(rw_hbm.at[pl.ds(tbase, half)], rw0v)
        pltpu.sync_copy(rw_hbm.at[pl.ds(S + tbase, half)], rw1v)
        pltpu.sync_copy(hid_hbm.at[pl.ds(tbase, half)], hidv)

        def body(i, carry):
            w0 = plsc.load_gather(rw0v, [zero + i])
            w1 = plsc.load_gather(rw1v, [zero + i])
            for j in range(D // LANES):
                sl = pl.ds(j * LANES, LANES)
                outv[i, sl] = (hidv[i, sl] + w0 * rows0[i, sl]
                               + w1 * rows1[i, sl])
            return carry

        lax.fori_loop(0, half, body, 0)
        pltpu.sync_copy(outv, out_hbm.at[pl.ds(tbase, half)])


def _combine(hidden, ys, inv, rw_flat):
    mesh = plsc.VectorSubcoreMesh(core_axis_name="c", subcore_axis_name="s")
    half = TPW // 2
    f = pl.kernel(
        _combine_body,
        out_type=jax.ShapeDtypeStruct((S, D), jnp.float32),
        mesh=mesh,
        compiler_params=pltpu.CompilerParams(needs_layout_passes=False),
        scratch_types=[
            pltpu.VMEM((half,), jnp.int32),
            pltpu.VMEM((half,), jnp.int32),
            pltpu.VMEM((half,), jnp.float32),
            pltpu.VMEM((half,), jnp.float32),
            pltpu.VMEM((half, D), jnp.float32),
            pltpu.VMEM((half, D), jnp.float32),
            pltpu.VMEM((half, D), jnp.float32),
            pltpu.VMEM((half, D), jnp.float32),
            pltpu.SemaphoreType.DMA,
        ],
    )
    return f(hidden, ys, inv, rw_flat)


def kernel(hidden_states, attention_mask, position_ids, ln1_w, ln2_w,
           wq, wk, wv, wo, gate_w, w1, w3, w2):
    x = hidden_states.reshape(S, D)

    inv_freq = 1.0 / (THETA ** (jnp.arange(0, HD, 2, dtype=jnp.float32) / HD))
    pos = position_ids.reshape(S).astype(jnp.float32)
    freqs = pos[:, None] * inv_freq[None, :]
    emb = jnp.concatenate([freqs, freqs], axis=-1)
    cos, sin = jnp.cos(emb), jnp.sin(emb)
    cos_q = jnp.tile(cos, (1, H))
    sin_q = jnp.tile(sin, (1, H))
    cos_k = jnp.tile(cos, (1, KVH))
    sin_k = jnp.tile(sin, (1, KVH))

    wq_b = wq.astype(jnp.bfloat16)
    wk_b = wk.astype(jnp.bfloat16)
    wv_b = wv.astype(jnp.bfloat16)
    wo_b = wo.astype(jnp.bfloat16)
    w1_b = w1.astype(jnp.bfloat16)
    w3_b = w3.astype(jnp.bfloat16)
    w2_b = w2.astype(jnp.bfloat16)

    q, k, v = _qkv(x, ln1_w.reshape(1, D), wq_b, wk_b, wv_b,
                   cos_q, sin_q, cos_k, sin_k)

    qh = q.reshape(S, H, HD).transpose(1, 0, 2)
    kh = k.reshape(S, KVH, HD).transpose(1, 0, 2)
    vh = v.reshape(S, KVH, HD).transpose(1, 0, 2)
    attn = _attention(qh, kh, vh)
    attn2 = attn.transpose(1, 0, 2).reshape(S, H * HD)

    hidden, x2f, sel, rw, cnt = _post_attn(x, attn2, wo_b,
                                           ln2_w.reshape(1, D), gate_w)

    sel_flat = sel.T.reshape(NP)
    rw_flat = rw.T.reshape(NP)
    xs, inv, bexp, nrows = _dispatch(sel_flat, cnt, x2f)
    ys = _gmm(bexp, nrows, xs, w1_b, w3_b, w2_b)
    out = _combine(hidden, ys, inv, rw_flat)
    return out.reshape(B, S, D)


# speedup vs baseline: 1.4525x; 1.2290x over previous
import functools
import math

import jax
import jax.numpy as jnp
import numpy as np
from jax import lax
from jax.experimental import pallas as pl
from jax.experimental.pallas import tpu as pltpu
from jax.experimental.pallas import tpu_sc as plsc

B, S, D = 1, 2048, 768
H, KVH, HD = 12, 4, 64
E, TOPK, F = 8, 2, 2048
EPS, THETA = 1e-6, 10000.0
REP = H // KVH

SB = 512
NSB = S // SB
BQ = 512
BK = 1024
NEG = -1e30

NW = 32
TPW = S // NW
PPW = 2 * TPW
NP = 2 * S
BLK = 128
NB = NP // BLK + E
PP = NB * BLK
NBP = 48
LANES = 16


def _rms(x, w):
    var = jnp.mean(jnp.square(x), axis=-1, keepdims=True)
    return (x * jax.lax.rsqrt(var + EPS)) * w


def _rot_half_heads(x):
    half = HD // 2
    s = jnp.concatenate([x[:, half:], x[:, :half]], axis=1)
    t = jnp.concatenate([x[:, -half:], x[:, :-half]], axis=1)
    lane = jax.lax.broadcasted_iota(jnp.int32, x.shape, 1)
    first = (lane % HD) < half
    return jnp.where(first, -s, t)


def _qkv_kernel(x_ref, ln1_ref, wq_ref, wk_ref, wv_ref, cq_ref, sq_ref,
                ck_ref, sk_ref, q_ref, k_ref, v_ref):
    h = _rms(x_ref[...], ln1_ref[...])
    hb = h.astype(jnp.bfloat16)
    q = jnp.dot(hb, wq_ref[...], preferred_element_type=jnp.float32)
    k = jnp.dot(hb, wk_ref[...], preferred_element_type=jnp.float32)
    v = jnp.dot(hb, wv_ref[...], preferred_element_type=jnp.float32)
    q = q * cq_ref[...] + _rot_half_heads(q) * sq_ref[...]
    k = k * ck_ref[...] + _rot_half_heads(k) * sk_ref[...]
    q_ref[...] = q.astype(jnp.bfloat16)
    k_ref[...] = k.astype(jnp.bfloat16)
    v_ref[...] = v.astype(jnp.bfloat16)


def _qkv(x, ln1_w, wq, wk, wv, cos_q, sin_q, cos_k, sin_k):
    return pl.pallas_call(
        _qkv_kernel,
        grid=(NSB,),
        in_specs=[
            pl.BlockSpec((SB, D), lambda i: (i, 0)),
            pl.BlockSpec((1, D), lambda i: (0, 0)),
            pl.BlockSpec((D, H * HD), lambda i: (0, 0)),
            pl.BlockSpec((D, KVH * HD), lambda i: (0, 0)),
            pl.BlockSpec((D, KVH * HD), lambda i: (0, 0)),
            pl.BlockSpec((SB, H * HD), lambda i: (i, 0)),
            pl.BlockSpec((SB, H * HD), lambda i: (i, 0)),
            pl.BlockSpec((SB, KVH * HD), lambda i: (i, 0)),
            pl.BlockSpec((SB, KVH * HD), lambda i: (i, 0)),
        ],
        out_specs=[
            pl.BlockSpec((SB, H * HD), lambda i: (i, 0)),
            pl.BlockSpec((SB, KVH * HD), lambda i: (i, 0)),
            pl.BlockSpec((SB, KVH * HD), lambda i: (i, 0)),
        ],
        out_shape=[
            jax.ShapeDtypeStruct((S, H * HD), jnp.bfloat16),
            jax.ShapeDtypeStruct((S, KVH * HD), jnp.bfloat16),
            jax.ShapeDtypeStruct((S, KVH * HD), jnp.bfloat16),
        ],
    )(x, ln1_w, wq, wk, wv, cos_q, sin_q, cos_k, sin_k)


MQ = REP * BQ


def _attn_kernel(q_ref, k_ref, v_ref, o_ref, acc_ref, m_ref, l_ref):
    qb = pl.program_id(1)
    kb = pl.program_id(2)

    @pl.when(kb == 0)
    def _init():
        acc_ref[...] = jnp.zeros_like(acc_ref)
        m_ref[...] = jnp.full_like(m_ref, NEG)
        l_ref[...] = jnp.zeros_like(l_ref)

    @pl.when(kb * BK <= qb * BQ + BQ - 1)
    def _compute():
        q = q_ref[0, 0]
        k = k_ref[0]
        s = jax.lax.dot_general(q, k, (((1,), (1,)), ((), ())),
                                preferred_element_type=jnp.float32)
        s = s * (1.0 / math.sqrt(HD))
        r = jax.lax.broadcasted_iota(jnp.int32, (MQ, BK), 0) % BQ
        row = qb * BQ + r
        col = kb * BK + jax.lax.broadcasted_iota(jnp.int32, (MQ, BK), 1)
        s = jnp.where(row >= col, s, NEG)
        m_prev = m_ref[...]
        m_cur = jnp.max(s, axis=-1, keepdims=True)
        m_new = jnp.maximum(m_prev, m_cur)
        p = jnp.exp(s - m_new)
        alpha = jnp.exp(m_prev - m_new)
        l_ref[...] = l_ref[...] * alpha + jnp.sum(p, axis=-1, keepdims=True)
        acc_ref[...] = acc_ref[...] * alpha + jnp.dot(
            p.astype(jnp.bfloat16), v_ref[0],
            preferred_element_type=jnp.float32)
        m_ref[...] = m_new

    @pl.when(kb == pl.num_programs(2) - 1)
    def _final():
        o_ref[0, 0] = (acc_ref[...] / l_ref[...]).astype(jnp.bfloat16)


def _attention(q, k, v):
    nqb = S // BQ
    return pl.pallas_call(
        _attn_kernel,
        grid=(KVH, nqb, S // BK),
        in_specs=[
            pl.BlockSpec((1, 1, MQ, HD), lambda g, i, j: (g, i, 0, 0)),
            pl.BlockSpec((1, BK, HD), lambda g, i, j: (g, j, 0)),
            pl.BlockSpec((1, BK, HD), lambda g, i, j: (g, j, 0)),
        ],
        out_specs=pl.BlockSpec((1, 1, MQ, HD), lambda g, i, j: (g, i, 0, 0)),
        out_shape=jax.ShapeDtypeStruct((KVH, nqb, MQ, HD), jnp.bfloat16),
        scratch_shapes=[
            pltpu.VMEM((MQ, HD), jnp.float32),
            pltpu.VMEM((MQ, 1), jnp.float32),
            pltpu.VMEM((MQ, 1), jnp.float32),
        ],
    )(q, k, v)


def _post_kernel(x_ref, attn_ref, wo_ref, ln2_ref, gate_ref,
                 hid_ref, x2_ref, sel_ref, rw_ref, cnt_ref):
    ao = jnp.dot(attn_ref[...], wo_ref[...],
                 preferred_element_type=jnp.float32)
    hid = x_ref[...] + ao
    hid_ref[...] = hid
    x2 = _rms(hid, ln2_ref[...])
    x2_ref[...] = x2
    logits = jax.lax.dot_general(
        x2, gate_ref[...], (((1,), (0,)), ((), ())),
        preferred_element_type=jnp.float32,
        precision=jax.lax.Precision.HIGHEST)
    mx = jnp.max(logits, axis=-1, keepdims=True)
    p = jnp.exp(logits - mx)
    p = p / jnp.sum(p, axis=-1, keepdims=True)
    lane = jax.lax.broadcasted_iota(jnp.int32, p.shape, 1)
    m1 = jnp.max(p, axis=-1, keepdims=True)
    e1 = jnp.min(jnp.where(p == m1, lane, E), axis=-1, keepdims=True)
    p2 = jnp.where(lane == e1, -1.0, p)
    m2 = jnp.max(p2, axis=-1, keepdims=True)
    e2 = jnp.min(jnp.where(p2 == m2, lane, E), axis=-1, keepdims=True)
    denom = m1 + m2
    sel_ref[...] = jnp.concatenate([e1, e2], axis=1)
    rw_ref[...] = jnp.concatenate([m1 / denom, m2 / denom], axis=1)
    lane16 = jax.lax.broadcasted_iota(jnp.int32, (SB, LANES), 1)
    oh = ((lane16 == e1).astype(jnp.bfloat16)
          + (lane16 == e2).astype(jnp.bfloat16))
    nch = SB // TPW
    ar = jax.lax.broadcasted_iota(jnp.int32, (nch, SB), 0)
    ac = jax.lax.broadcasted_iota(jnp.int32, (nch, SB), 1) // TPW
    seg = (ar == ac).astype(jnp.bfloat16)
    cnt_ref[...] = jnp.dot(seg, oh,
                           preferred_element_type=jnp.float32).astype(jnp.int32)


def _post_attn(x, attn, wo, ln2_w, gate_w):
    nch = SB // TPW
    return pl.pallas_call(
        _post_kernel,
        grid=(NSB,),
        in_specs=[
            pl.BlockSpec((SB, D), lambda i: (i, 0)),
            pl.BlockSpec((SB, H * HD), lambda i: (i, 0)),
            pl.BlockSpec((H * HD, D), lambda i: (0, 0)),
            pl.BlockSpec((1, D), lambda i: (0, 0)),
            pl.BlockSpec((D, E), lambda i: (0, 0)),
        ],
        out_specs=[
            pl.BlockSpec((SB, D), lambda i: (i, 0)),
            pl.BlockSpec((SB, D), lambda i: (i, 0)),
            pl.BlockSpec((SB, TOPK), lambda i: (i, 0)),
            pl.BlockSpec((SB, TOPK), lambda i: (i, 0)),
            pl.BlockSpec((nch, LANES), lambda i: (i, 0)),
        ],
        out_shape=[
            jax.ShapeDtypeStruct((S, D), jnp.float32),
            jax.ShapeDtypeStruct((S, D), jnp.float32),
            jax.ShapeDtypeStruct((S, TOPK), jnp.int32),
            jax.ShapeDtypeStruct((S, TOPK), jnp.float32),
            jax.ShapeDtypeStruct((NW, LANES), jnp.int32),
        ],
    )(x, attn, wo, ln2_w, gate_w)


def _bc(vec, lane):
    idx = jnp.zeros((LANES,), jnp.int32) + lane
    return vec.at[idx].get(mode="promise_in_bounds")


def _dispatch_body(sel_hbm, cnt_hbm, x2_hbm,
                   xs_hbm, inv_hbm, bexp_hbm, nrows_hbm,
                   selv, k0v, k1v, cntv, t1v, t2v, x2rows, sem):
    nc = 2
    wid = lax.axis_index("s") * nc + lax.axis_index("c")
    iota = jnp.arange(LANES, dtype=jnp.int32)
    zero = jnp.zeros((LANES,), jnp.int32)
    widv = zero + wid

    pltpu.sync_copy(cnt_hbm, cntv)
    tot = jnp.zeros((LANES,), jnp.int32)
    pfx = jnp.zeros((LANES,), jnp.int32)
    for ch in range(NW):
        row = cntv[ch]
        tot = tot + row
        pfx = pfx + jnp.where((zero + ch) < widv, row, 0)
    nb = (tot + (BLK - 1)) // BLK
    blk_end = plsc.cumsum(nb)
    blk_start = blk_end - nb
    roff = blk_start * BLK + pfx

    pltpu.sync_copy(sel_hbm.at[pl.ds(wid * TPW, TPW)],
                    selv.at[pl.ds(0, TPW)])
    pltpu.sync_copy(sel_hbm.at[pl.ds(S + wid * TPW, TPW)],
                    selv.at[pl.ds(TPW, TPW)])
    offs = [_bc(roff, e) for e in range(E)]
    runs = [jnp.zeros((LANES,), jnp.int32) for _ in range(E)]
    nchunk = PPW // LANES
    for c in range(nchunk):
        v = selv[pl.ds(c * LANES, LANES)]
        dest = jnp.zeros((LANES,), jnp.int32)
        for e in range(E):
            m = v == e
            csum = plsc.cumsum(jnp.where(m, 1, 0))
            dest = jnp.where(m, offs[e] + runs[e] + csum - 1, dest)
            runs[e] = runs[e] + plsc.all_reduce_population_count(m)
        if c < nchunk // 2:
            k0v[pl.ds(c * LANES, LANES)] = dest
        else:
            k1v[pl.ds((c - nchunk // 2) * LANES, LANES)] = dest
    pltpu.sync_copy(k0v, inv_hbm.at[pl.ds(wid * TPW, TPW)])
    pltpu.sync_copy(k1v, inv_hbm.at[pl.ds(S + wid * TPW, TPW)])

    pltpu.sync_copy(x2_hbm.at[pl.ds(wid * TPW, TPW)], x2rows)
    pltpu.async_copy(x2rows, xs_hbm.at[k0v], sem).wait()
    pltpu.async_copy(x2rows, xs_hbm.at[k1v], sem).wait()

    @pl.when(wid < NBP // LANES)
    def _tables():
        b_vec = zero + wid * LANES + iota
        be = jnp.zeros((LANES,), jnp.int32)
        for e in range(E):
            be = be + jnp.where(b_vec >= _bc(blk_end, e), 1, 0)
        t1v[...] = jnp.minimum(be, E - 1)
        nr = jnp.zeros((LANES,), jnp.int32)
        for e in range(E):
            rem = (_bc(tot, e) - (b_vec - _bc(blk_start, e)) * BLK)
            nr = jnp.where(be == e, jnp.clip(rem, 0, BLK), nr)
        t2v[...] = nr
        pltpu.sync_copy(t1v, bexp_hbm.at[pl.ds(wid * LANES, LANES)])
        pltpu.sync_copy(t2v, nrows_hbm.at[pl.ds(wid * LANES, LANES)])


def _dispatch(sel_flat, cnt, x2f):
    mesh = plsc.VectorSubcoreMesh(core_axis_name="c", subcore_axis_name="s")
    f = pl.kernel(
        _dispatch_body,
        out_type=[
            jax.ShapeDtypeStruct((PP, D), jnp.float32),
            jax.ShapeDtypeStruct((NP,), jnp.int32),
            jax.ShapeDtypeStruct((NBP,), jnp.int32),
            jax.ShapeDtypeStruct((NBP,), jnp.int32),
        ],
        mesh=mesh,
        compiler_params=pltpu.CompilerParams(needs_layout_passes=False),
        scratch_types=[
            pltpu.VMEM((PPW,), jnp.int32),
            pltpu.VMEM((TPW,), jnp.int32),
            pltpu.VMEM((TPW,), jnp.int32),
            pltpu.VMEM((NW, LANES), jnp.int32),
            pltpu.VMEM((LANES,), jnp.int32),
            pltpu.VMEM((LANES,), jnp.int32),
            pltpu.VMEM((TPW, D), jnp.float32),
            pltpu.SemaphoreType.DMA,
        ],
    )
    return f(sel_flat, cnt, x2f)


def _gmm_kernel(bexp_ref, nrows_ref, xs_ref, w1_ref, w3_ref, w2_ref, ys_ref):
    b = pl.program_id(0)
    n = nrows_ref[b]

    @pl.when(n > 0)
    def _compute():
        rowi = jax.lax.broadcasted_iota(jnp.int32, (BLK, D), 0)
        x = jnp.where(rowi < n, xs_ref[...], 0.0).astype(jnp.bfloat16)
        a = jnp.dot(x, w1_ref[0], preferred_element_type=jnp.float32)
        t = jnp.dot(x, w3_ref[0], preferred_element_type=jnp.float32)
        g = (a * jax.lax.logistic(a)) * t
        ys_ref[...] = jnp.dot(g.astype(jnp.bfloat16), w2_ref[0],
                              preferred_element_type=jnp.float32)


def _gmm(bexp, nrows, xs, w1, w3, w2):
    grid_spec = pltpu.PrefetchScalarGridSpec(
        num_scalar_prefetch=2,
        grid=(NB,),
        in_specs=[
            pl.BlockSpec((BLK, D), lambda b, be, nr: (b, 0)),
            pl.BlockSpec((1, D, F), lambda b, be, nr: (be[b], 0, 0)),
            pl.BlockSpec((1, D, F), lambda b, be, nr: (be[b], 0, 0)),
            pl.BlockSpec((1, F, D), lambda b, be, nr: (be[b], 0, 0)),
        ],
        out_specs=pl.BlockSpec((BLK, D), lambda b, be, nr: (b, 0)),
    )
    return pl.pallas_call(
        _gmm_kernel,
        grid_spec=grid_spec,
        out_shape=jax.ShapeDtypeStruct((PP, D), jnp.float32),
    )(bexp, nrows, xs, w1, w3, w2)


def _combine_body(hid_hbm, ys_hbm, inv_hbm, rw_hbm, out_hbm,
                  i0v, i1v, rw0v, rw1v, rows0, rows1, hidv, outv, sem):
    nc = 2
    wid = lax.axis_index("s") * nc + lax.axis_index("c")
    zero = jnp.zeros((LANES,), jnp.int32)
    half = TPW // 2
    for h in range(2):
        tbase = wid * TPW + h * half
        pltpu.sync_copy(inv_hbm.at[pl.ds(tbase, half)], i0v)
        pltpu.sync_copy(inv_hbm.at[pl.ds(S + tbase, half)], i1v)
        pltpu.async_copy(ys_hbm.at[i0v], rows0, sem).wait()
        pltpu.async_copy(ys_hbm.at[i1v], rows1, sem).wait()
        pltpu.sync_copy(rw_hbm.at[pl.ds(tbase, half)], rw0v)
        pltpu.sync_copy(rw_hbm.at[pl.ds(S + tbase, half)], rw1v)
        pltpu.sync_copy(hid_hbm.at[pl.ds(tbase, half)], hidv)

        def body(i, carry):
            w0 = plsc.load_gather(rw0v, [zero + i])
            w1 = plsc.load_gather(rw1v, [zero + i])
            for j in range(D // LANES):
                sl = pl.ds(j * LANES, LANES)
                outv[i, sl] = (hidv[i, sl] + w0 * rows0[i, sl]
                               + w1 * rows1[i, sl])
            return carry

        lax.fori_loop(0, half, body, 0)
        pltpu.sync_copy(outv, out_hbm.at[pl.ds(tbase, half)])


def _combine(hidden, ys, inv, rw_flat):
    mesh = plsc.VectorSubcoreMesh(core_axis_name="c", subcore_axis_name="s")
    half = TPW // 2
    f = pl.kernel(
        _combine_body,
        out_type=jax.ShapeDtypeStruct((S, D), jnp.float32),
        mesh=mesh,
        compiler_params=pltpu.CompilerParams(needs_layout_passes=False),
        scratch_types=[
            pltpu.VMEM((half,), jnp.int32),
            pltpu.VMEM((half,), jnp.int32),
            pltpu.VMEM((half,), jnp.float32),
            pltpu.VMEM((half,), jnp.float32),
            pltpu.VMEM((half, D), jnp.float32),
            pltpu.VMEM((half, D), jnp.float32),
            pltpu.VMEM((half, D), jnp.float32),
            pltpu.VMEM((half, D), jnp.float32),
            pltpu.SemaphoreType.DMA,
        ],
    )
    return f(hidden, ys, inv, rw_flat)


def kernel(hidden_states, attention_mask, position_ids, ln1_w, ln2_w,
           wq, wk, wv, wo, gate_w, w1, w3, w2):
    x = hidden_states.reshape(S, D)

    inv_freq = 1.0 / (THETA ** (jnp.arange(0, HD, 2, dtype=jnp.float32) / HD))
    pos = position_ids.reshape(S).astype(jnp.float32)
    freqs = pos[:, None] * inv_freq[None, :]
    emb = jnp.concatenate([freqs, freqs], axis=-1)
    cos, sin = jnp.cos(emb), jnp.sin(emb)
    cos_q = jnp.tile(cos, (1, H))
    sin_q = jnp.tile(sin, (1, H))
    cos_k = jnp.tile(cos, (1, KVH))
    sin_k = jnp.tile(sin, (1, KVH))

    wq_b = wq.astype(jnp.bfloat16)
    wk_b = wk.astype(jnp.bfloat16)
    wv_b = wv.astype(jnp.bfloat16)
    wo_b = wo.astype(jnp.bfloat16)
    w1_b = w1.astype(jnp.bfloat16)
    w3_b = w3.astype(jnp.bfloat16)
    w2_b = w2.astype(jnp.bfloat16)

    q, k, v = _qkv(x, ln1_w.reshape(1, D), wq_b, wk_b, wv_b,
                   cos_q, sin_q, cos_k, sin_k)

    nqb = S // BQ
    qh = (q.reshape(nqb, BQ, KVH, REP, HD)
          .transpose(2, 0, 3, 1, 4).reshape(KVH, nqb, MQ, HD))
    kh = k.reshape(S, KVH, HD).transpose(1, 0, 2)
    vh = v.reshape(S, KVH, HD).transpose(1, 0, 2)
    attn = _attention(qh, kh, vh)
    attn2 = (attn.reshape(KVH, nqb, REP, BQ, HD)
             .transpose(1, 3, 0, 2, 4).reshape(S, H * HD))

    hidden, x2f, sel, rw, cnt = _post_attn(x, attn2, wo_b,
                                           ln2_w.reshape(1, D), gate_w)

    sel_flat = sel.T.reshape(NP)
    rw_flat = rw.T.reshape(NP)
    xs, inv, bexp, nrows = _dispatch(sel_flat, cnt, x2f)
    ys = _gmm(bexp, nrows, xs, w1_b, w3_b, w2_b)
    out = _combine(hidden, ys, inv, rw_flat)
    return out.reshape(B, S, D)


# speedup vs baseline: 1.4938x; 1.0285x over previous
import functools
import math

import jax
import jax.numpy as jnp
import numpy as np
from jax import lax
from jax.experimental import pallas as pl
from jax.experimental.pallas import tpu as pltpu
from jax.experimental.pallas import tpu_sc as plsc

B, S, D = 1, 2048, 768
H, KVH, HD = 12, 4, 64
E, TOPK, F = 8, 2, 2048
EPS, THETA = 1e-6, 10000.0
REP = H // KVH

SB = 512
NSB = S // SB
BQ = 512
BK = 1024
NEG = -1e30

NW = 32
TPW = S // NW
PPW = 2 * TPW
NP = 2 * S
BLK = 256
NB = NP // BLK + E
PP = NB * BLK
NBP = 32
LANES = 16


def _rms(x, w):
    var = jnp.mean(jnp.square(x), axis=-1, keepdims=True)
    return (x * jax.lax.rsqrt(var + EPS)) * w


def _rot_half_heads(x):
    half = HD // 2
    s = jnp.concatenate([x[:, half:], x[:, :half]], axis=1)
    t = jnp.concatenate([x[:, -half:], x[:, :-half]], axis=1)
    lane = jax.lax.broadcasted_iota(jnp.int32, x.shape, 1)
    first = (lane % HD) < half
    return jnp.where(first, -s, t)


def _qkv_kernel(x_ref, ln1_ref, wq_ref, wk_ref, wv_ref, cq_ref, sq_ref,
                ck_ref, sk_ref, q_ref, k_ref, v_ref):
    h = _rms(x_ref[...], ln1_ref[...])
    hb = h.astype(jnp.bfloat16)
    q = jnp.dot(hb, wq_ref[...], preferred_element_type=jnp.float32)
    k = jnp.dot(hb, wk_ref[...], preferred_element_type=jnp.float32)
    v = jnp.dot(hb, wv_ref[...], preferred_element_type=jnp.float32)
    q = q * cq_ref[...] + _rot_half_heads(q) * sq_ref[...]
    k = k * ck_ref[...] + _rot_half_heads(k) * sk_ref[...]
    q_ref[...] = q.astype(jnp.bfloat16)
    k_ref[...] = k.astype(jnp.bfloat16)
    v_ref[...] = v.astype(jnp.bfloat16)


def _qkv(x, ln1_w, wq, wk, wv, cos_q, sin_q, cos_k, sin_k):
    return pl.pallas_call(
        _qkv_kernel,
        grid=(NSB,),
        in_specs=[
            pl.BlockSpec((SB, D), lambda i: (i, 0)),
            pl.BlockSpec((1, D), lambda i: (0, 0)),
            pl.BlockSpec((D, H * HD), lambda i: (0, 0)),
            pl.BlockSpec((D, KVH * HD), lambda i: (0, 0)),
            pl.BlockSpec((D, KVH * HD), lambda i: (0, 0)),
            pl.BlockSpec((SB, H * HD), lambda i: (i, 0)),
            pl.BlockSpec((SB, H * HD), lambda i: (i, 0)),
            pl.BlockSpec((SB, KVH * HD), lambda i: (i, 0)),
            pl.BlockSpec((SB, KVH * HD), lambda i: (i, 0)),
        ],
        out_specs=[
            pl.BlockSpec((SB, H * HD), lambda i: (i, 0)),
            pl.BlockSpec((SB, KVH * HD), lambda i: (i, 0)),
            pl.BlockSpec((SB, KVH * HD), lambda i: (i, 0)),
        ],
        out_shape=[
            jax.ShapeDtypeStruct((S, H * HD), jnp.bfloat16),
            jax.ShapeDtypeStruct((S, KVH * HD), jnp.bfloat16),
            jax.ShapeDtypeStruct((S, KVH * HD), jnp.bfloat16),
        ],
    )(x, ln1_w, wq, wk, wv, cos_q, sin_q, cos_k, sin_k)


MQ = REP * BQ


def _attn_kernel(q_ref, k_ref, v_ref, o_ref, acc_ref, m_ref, l_ref):
    qb = pl.program_id(1)
    kb = pl.program_id(2)

    @pl.when(kb == 0)
    def _init():
        acc_ref[...] = jnp.zeros_like(acc_ref)
        m_ref[...] = jnp.full_like(m_ref, NEG)
        l_ref[...] = jnp.zeros_like(l_ref)

    @pl.when(kb * BK <= qb * BQ + BQ - 1)
    def _compute():
        q = q_ref[0, 0]
        k = k_ref[0]
        s = jax.lax.dot_general(q, k, (((1,), (1,)), ((), ())),
                                preferred_element_type=jnp.float32)
        s = s * (1.0 / math.sqrt(HD))
        r = jax.lax.broadcasted_iota(jnp.int32, (MQ, BK), 0) % BQ
        row = qb * BQ + r
        col = kb * BK + jax.lax.broadcasted_iota(jnp.int32, (MQ, BK), 1)
        s = jnp.where(row >= col, s, NEG)
        m_prev = m_ref[...]
        m_cur = jnp.max(s, axis=-1, keepdims=True)
        m_new = jnp.maximum(m_prev, m_cur)
        p = jnp.exp(s - m_new)
        alpha = jnp.exp(m_prev - m_new)
        l_ref[...] = l_ref[...] * alpha + jnp.sum(p, axis=-1, keepdims=True)
        acc_ref[...] = acc_ref[...] * alpha + jnp.dot(
            p.astype(jnp.bfloat16), v_ref[0],
            preferred_element_type=jnp.float32)
        m_ref[...] = m_new

    @pl.when(kb == pl.num_programs(2) - 1)
    def _final():
        o_ref[0, 0] = (acc_ref[...] / l_ref[...]).astype(jnp.bfloat16)


def _attention(q, k, v):
    nqb = S // BQ
    return pl.pallas_call(
        _attn_kernel,
        grid=(KVH, nqb, S // BK),
        in_specs=[
            pl.BlockSpec((1, 1, MQ, HD), lambda g, i, j: (g, i, 0, 0)),
            pl.BlockSpec((1, BK, HD), lambda g, i, j: (g, j, 0)),
            pl.BlockSpec((1, BK, HD), lambda g, i, j: (g, j, 0)),
        ],
        out_specs=pl.BlockSpec((1, 1, MQ, HD), lambda g, i, j: (g, i, 0, 0)),
        out_shape=jax.ShapeDtypeStruct((KVH, nqb, MQ, HD), jnp.bfloat16),
        scratch_shapes=[
            pltpu.VMEM((MQ, HD), jnp.float32),
            pltpu.VMEM((MQ, 1), jnp.float32),
            pltpu.VMEM((MQ, 1), jnp.float32),
        ],
    )(q, k, v)


def _post_kernel(x_ref, attn_ref, wo_ref, ln2_ref, gate_ref,
                 hid_ref, x2_ref, sel_ref, rw_ref, cnt_ref):
    ao = jnp.dot(attn_ref[...], wo_ref[...],
                 preferred_element_type=jnp.float32)
    hid = x_ref[...] + ao
    hid_ref[...] = hid
    x2 = _rms(hid, ln2_ref[...])
    x2_ref[...] = x2
    logits = jax.lax.dot_general(
        x2, gate_ref[...], (((1,), (0,)), ((), ())),
        preferred_element_type=jnp.float32,
        precision=jax.lax.Precision.HIGHEST)
    mx = jnp.max(logits, axis=-1, keepdims=True)
    p = jnp.exp(logits - mx)
    p = p / jnp.sum(p, axis=-1, keepdims=True)
    lane = jax.lax.broadcasted_iota(jnp.int32, p.shape, 1)
    m1 = jnp.max(p, axis=-1, keepdims=True)
    e1 = jnp.min(jnp.where(p == m1, lane, E), axis=-1, keepdims=True)
    p2 = jnp.where(lane == e1, -1.0, p)
    m2 = jnp.max(p2, axis=-1, keepdims=True)
    e2 = jnp.min(jnp.where(p2 == m2, lane, E), axis=-1, keepdims=True)
    denom = m1 + m2
    sel_ref[...] = jnp.concatenate([e1, e2], axis=1)
    rw_ref[...] = jnp.concatenate([m1 / denom, m2 / denom], axis=1)
    lane16 = jax.lax.broadcasted_iota(jnp.int32, (SB, LANES), 1)
    oh = ((lane16 == e1).astype(jnp.bfloat16)
          + (lane16 == e2).astype(jnp.bfloat16))
    nch = SB // TPW
    ar = jax.lax.broadcasted_iota(jnp.int32, (nch, SB), 0)
    ac = jax.lax.broadcasted_iota(jnp.int32, (nch, SB), 1) // TPW
    seg = (ar == ac).astype(jnp.bfloat16)
    cnt_ref[...] = jnp.dot(seg, oh,
                           preferred_element_type=jnp.float32).astype(jnp.int32)


def _post_attn(x, attn, wo, ln2_w, gate_w):
    nch = SB // TPW
    return pl.pallas_call(
        _post_kernel,
        grid=(NSB,),
        in_specs=[
            pl.BlockSpec((SB, D), lambda i: (i, 0)),
            pl.BlockSpec((SB, H * HD), lambda i: (i, 0)),
            pl.BlockSpec((H * HD, D), lambda i: (0, 0)),
            pl.BlockSpec((1, D), lambda i: (0, 0)),
            pl.BlockSpec((D, E), lambda i: (0, 0)),
        ],
        out_specs=[
            pl.BlockSpec((SB, D), lambda i: (i, 0)),
            pl.BlockSpec((SB, D), lambda i: (i, 0)),
            pl.BlockSpec((SB, TOPK), lambda i: (i, 0)),
            pl.BlockSpec((SB, TOPK), lambda i: (i, 0)),
            pl.BlockSpec((nch, LANES), lambda i: (i, 0)),
        ],
        out_shape=[
            jax.ShapeDtypeStruct((S, D), jnp.float32),
            jax.ShapeDtypeStruct((S, D), jnp.float32),
            jax.ShapeDtypeStruct((S, TOPK), jnp.int32),
            jax.ShapeDtypeStruct((S, TOPK), jnp.float32),
            jax.ShapeDtypeStruct((NW, LANES), jnp.int32),
        ],
    )(x, attn, wo, ln2_w, gate_w)


def _bc(vec, lane):
    idx = jnp.zeros((LANES,), jnp.int32) + lane
    return vec.at[idx].get(mode="promise_in_bounds")


def _dispatch_body(sel_hbm, cnt_hbm, x2_hbm,
                   xs_hbm, inv_hbm, bexp_hbm, nrows_hbm,
                   selv, k0v, k1v, cntv, t1v, t2v, x2rows, sem):
    nc = 2
    wid = lax.axis_index("s") * nc + lax.axis_index("c")
    iota = jnp.arange(LANES, dtype=jnp.int32)
    zero = jnp.zeros((LANES,), jnp.int32)
    widv = zero + wid

    pltpu.sync_copy(cnt_hbm, cntv)
    tot = jnp.zeros((LANES,), jnp.int32)
    pfx = jnp.zeros((LANES,), jnp.int32)
    for ch in range(NW):
        row = cntv[ch]
        tot = tot + row
        pfx = pfx + jnp.where((zero + ch) < widv, row, 0)
    nb = (tot + (BLK - 1)) // BLK
    blk_end = plsc.cumsum(nb)
    blk_start = blk_end - nb
    roff = blk_start * BLK + pfx

    pltpu.sync_copy(sel_hbm.at[pl.ds(wid * TPW, TPW)],
                    selv.at[pl.ds(0, TPW)])
    pltpu.sync_copy(sel_hbm.at[pl.ds(S + wid * TPW, TPW)],
                    selv.at[pl.ds(TPW, TPW)])
    offs = [_bc(roff, e) for e in range(E)]
    runs = [jnp.zeros((LANES,), jnp.int32) for _ in range(E)]
    nchunk = PPW // LANES
    for c in range(nchunk):
        v = selv[pl.ds(c * LANES, LANES)]
        dest = jnp.zeros((LANES,), jnp.int32)
        for e in range(E):
            m = v == e
            csum = plsc.cumsum(jnp.where(m, 1, 0))
            dest = jnp.where(m, offs[e] + runs[e] + csum - 1, dest)
            runs[e] = runs[e] + plsc.all_reduce_population_count(m)
        if c < nchunk // 2:
            k0v[pl.ds(c * LANES, LANES)] = dest
        else:
            k1v[pl.ds((c - nchunk // 2) * LANES, LANES)] = dest
    pltpu.sync_copy(k0v, inv_hbm.at[pl.ds(wid * TPW, TPW)])
    pltpu.sync_copy(k1v, inv_hbm.at[pl.ds(S + wid * TPW, TPW)])

    pltpu.sync_copy(x2_hbm.at[pl.ds(wid * TPW, TPW)], x2rows)
    pltpu.async_copy(x2rows, xs_hbm.at[k0v], sem).wait()
    pltpu.async_copy(x2rows, xs_hbm.at[k1v], sem).wait()

    @pl.when(wid < NBP // LANES)
    def _tables():
        b_vec = zero + wid * LANES + iota
        be = jnp.zeros((LANES,), jnp.int32)
        for e in range(E):
            be = be + jnp.where(b_vec >= _bc(blk_end, e), 1, 0)
        t1v[...] = jnp.minimum(be, E - 1)
        nr = jnp.zeros((LANES,), jnp.int32)
        for e in range(E):
            rem = (_bc(tot, e) - (b_vec - _bc(blk_start, e)) * BLK)
            nr = jnp.where(be == e, jnp.clip(rem, 0, BLK), nr)
        t2v[...] = nr
        pltpu.sync_copy(t1v, bexp_hbm.at[pl.ds(wid * LANES, LANES)])
        pltpu.sync_copy(t2v, nrows_hbm.at[pl.ds(wid * LANES, LANES)])


def _dispatch(sel_flat, cnt, x2f):
    mesh = plsc.VectorSubcoreMesh(core_axis_name="c", subcore_axis_name="s")
    f = pl.kernel(
        _dispatch_body,
        out_type=[
            jax.ShapeDtypeStruct((PP, D), jnp.float32),
            jax.ShapeDtypeStruct((NP,), jnp.int32),
            jax.ShapeDtypeStruct((NBP,), jnp.int32),
            jax.ShapeDtypeStruct((NBP,), jnp.int32),
        ],
        mesh=mesh,
        compiler_params=pltpu.CompilerParams(needs_layout_passes=False),
        scratch_types=[
            pltpu.VMEM((PPW,), jnp.int32),
            pltpu.VMEM((TPW,), jnp.int32),
            pltpu.VMEM((TPW,), jnp.int32),
            pltpu.VMEM((NW, LANES), jnp.int32),
            pltpu.VMEM((LANES,), jnp.int32),
            pltpu.VMEM((LANES,), jnp.int32),
            pltpu.VMEM((TPW, D), jnp.float32),
            pltpu.SemaphoreType.DMA,
        ],
    )
    return f(sel_flat, cnt, x2f)


def _gmm_kernel(bexp_ref, nrows_ref, xs_ref, w1_ref, w3_ref, w2_ref, ys_ref):
    b = pl.program_id(0)
    n = nrows_ref[b]

    @pl.when(n > 0)
    def _compute():
        rowi = jax.lax.broadcasted_iota(jnp.int32, (BLK, D), 0)
        x = jnp.where(rowi < n, xs_ref[...], 0.0).astype(jnp.bfloat16)
        a = jnp.dot(x, w1_ref[0], preferred_element_type=jnp.float32)
        t = jnp.dot(x, w3_ref[0], preferred_element_type=jnp.float32)
        g = (a * jax.lax.logistic(a)) * t
        ys_ref[...] = jnp.dot(g.astype(jnp.bfloat16), w2_ref[0],
                              preferred_element_type=jnp.float32)


def _gmm(bexp, nrows, xs, w1, w3, w2):
    grid_spec = pltpu.PrefetchScalarGridSpec(
        num_scalar_prefetch=2,
        grid=(NB,),
        in_specs=[
            pl.BlockSpec((BLK, D), lambda b, be, nr: (b, 0)),
            pl.BlockSpec((1, D, F), lambda b, be, nr: (be[b], 0, 0)),
            pl.BlockSpec((1, D, F), lambda b, be, nr: (be[b], 0, 0)),
            pl.BlockSpec((1, F, D), lambda b, be, nr: (be[b], 0, 0)),
        ],
        out_specs=pl.BlockSpec((BLK, D), lambda b, be, nr: (b, 0)),
    )
    return pl.pallas_call(
        _gmm_kernel,
        grid_spec=grid_spec,
        out_shape=jax.ShapeDtypeStruct((PP, D), jnp.float32),
    )(bexp, nrows, xs, w1, w3, w2)


def _combine_body(hid_hbm, ys_hbm, inv_hbm, rw_hbm, out_hbm,
                  i0v, i1v, rw0v, rw1v, rows0, rows1, hidv, outv, sem):
    nc = 2
    wid = lax.axis_index("s") * nc + lax.axis_index("c")
    zero = jnp.zeros((LANES,), jnp.int32)
    half = TPW // 2
    for h in range(2):
        tbase = wid * TPW + h * half
        pltpu.sync_copy(inv_hbm.at[pl.ds(tbase, half)], i0v)
        pltpu.sync_copy(inv_hbm.at[pl.ds(S + tbase, half)], i1v)
        pltpu.async_copy(ys_hbm.at[i0v], rows0, sem).wait()
        pltpu.async_copy(ys_hbm.at[i1v], rows1, sem).wait()
        pltpu.sync_copy(rw_hbm.at[pl.ds(tbase, half)], rw0v)
        pltpu.sync_copy(rw_hbm.at[pl.ds(S + tbase, half)], rw1v)
        pltpu.sync_copy(hid_hbm.at[pl.ds(tbase, half)], hidv)

        def body(i, carry):
            w0 = plsc.load_gather(rw0v, [zero + i])
            w1 = plsc.load_gather(rw1v, [zero + i])
            for j in range(D // LANES):
                sl = pl.ds(j * LANES, LANES)
                outv[i, sl] = (hidv[i, sl] + w0 * rows0[i, sl]
                               + w1 * rows1[i, sl])
            return carry

        lax.fori_loop(0, half, body, 0)
        pltpu.sync_copy(outv, out_hbm.at[pl.ds(tbase, half)])


def _combine(hidden, ys, inv, rw_flat):
    mesh = plsc.VectorSubcoreMesh(core_axis_name="c", subcore_axis_name="s")
    half = TPW // 2
    f = pl.kernel(
        _combine_body,
        out_type=jax.ShapeDtypeStruct((S, D), jnp.float32),
        mesh=mesh,
        compiler_params=pltpu.CompilerParams(needs_layout_passes=False),
        scratch_types=[
            pltpu.VMEM((half,), jnp.int32),
            pltpu.VMEM((half,), jnp.int32),
            pltpu.VMEM((half,), jnp.float32),
            pltpu.VMEM((half,), jnp.float32),
            pltpu.VMEM((half, D), jnp.float32),
            pltpu.VMEM((half, D), jnp.float32),
            pltpu.VMEM((half, D), jnp.float32),
            pltpu.VMEM((half, D), jnp.float32),
            pltpu.SemaphoreType.DMA,
        ],
    )
    return f(hidden, ys, inv, rw_flat)


def kernel(hidden_states, attention_mask, position_ids, ln1_w, ln2_w,
           wq, wk, wv, wo, gate_w, w1, w3, w2):
    x = hidden_states.reshape(S, D)

    inv_freq = 1.0 / (THETA ** (jnp.arange(0, HD, 2, dtype=jnp.float32) / HD))
    pos = position_ids.reshape(S).astype(jnp.float32)
    freqs = pos[:, None] * inv_freq[None, :]
    emb = jnp.concatenate([freqs, freqs], axis=-1)
    cos, sin = jnp.cos(emb), jnp.sin(emb)
    cos_q = jnp.tile(cos, (1, H))
    sin_q = jnp.tile(sin, (1, H))
    cos_k = jnp.tile(cos, (1, KVH))
    sin_k = jnp.tile(sin, (1, KVH))

    wq_b = wq.astype(jnp.bfloat16)
    wk_b = wk.astype(jnp.bfloat16)
    wv_b = wv.astype(jnp.bfloat16)
    wo_b = wo.astype(jnp.bfloat16)
    w1_b = w1.astype(jnp.bfloat16)
    w3_b = w3.astype(jnp.bfloat16)
    w2_b = w2.astype(jnp.bfloat16)

    q, k, v = _qkv(x, ln1_w.reshape(1, D), wq_b, wk_b, wv_b,
                   cos_q, sin_q, cos_k, sin_k)

    nqb = S // BQ
    qh = (q.reshape(nqb, BQ, KVH, REP, HD)
          .transpose(2, 0, 3, 1, 4).reshape(KVH, nqb, MQ, HD))
    kh = k.reshape(S, KVH, HD).transpose(1, 0, 2)
    vh = v.reshape(S, KVH, HD).transpose(1, 0, 2)
    attn = _attention(qh, kh, vh)
    attn2 = (attn.reshape(KVH, nqb, REP, BQ, HD)
             .transpose(1, 3, 0, 2, 4).reshape(S, H * HD))

    hidden, x2f, sel, rw, cnt = _post_attn(x, attn2, wo_b,
                                           ln2_w.reshape(1, D), gate_w)

    sel_flat = sel.T.reshape(NP)
    rw_flat = rw.T.reshape(NP)
    xs, inv, bexp, nrows = _dispatch(sel_flat, cnt, x2f)
    ys = _gmm(bexp, nrows, xs, w1_b, w3_b, w2_b)
    out = _combine(hidden, ys, inv, rw_flat)
    return out.reshape(B, S, D)


# speedup vs baseline: 1.5024x; 1.0057x over previous
import functools
import math

import jax
import jax.numpy as jnp
import numpy as np
from jax import lax
from jax.experimental import pallas as pl
from jax.experimental.pallas import tpu as pltpu
from jax.experimental.pallas import tpu_sc as plsc

B, S, D = 1, 2048, 768
H, KVH, HD = 12, 4, 64
E, TOPK, F = 8, 2, 2048
EPS, THETA = 1e-6, 10000.0
REP = H // KVH

SB = 512
NSB = S // SB
BQ = 1024
BK = 1024
NEG = -1e30

NW = 32
TPW = S // NW
PPW = 2 * TPW
NP = 2 * S
BLK = 256
NB = NP // BLK + E
PP = NB * BLK
NBP = 32
LANES = 16


def _rms(x, w):
    var = jnp.mean(jnp.square(x), axis=-1, keepdims=True)
    return (x * jax.lax.rsqrt(var + EPS)) * w


def _rot_half_heads(x):
    half = HD // 2
    s = jnp.concatenate([x[:, half:], x[:, :half]], axis=1)
    t = jnp.concatenate([x[:, -half:], x[:, :-half]], axis=1)
    lane = jax.lax.broadcasted_iota(jnp.int32, x.shape, 1)
    first = (lane % HD) < half
    return jnp.where(first, -s, t)


def _qkv_kernel(x_ref, ln1_ref, wq_ref, wk_ref, wv_ref, cq_ref, sq_ref,
                ck_ref, sk_ref, q_ref, k_ref, v_ref):
    h = _rms(x_ref[...], ln1_ref[...])
    hb = h.astype(jnp.bfloat16)
    q = jnp.dot(hb, wq_ref[...], preferred_element_type=jnp.float32)
    k = jnp.dot(hb, wk_ref[...], preferred_element_type=jnp.float32)
    v = jnp.dot(hb, wv_ref[...], preferred_element_type=jnp.float32)
    q = q * cq_ref[...] + _rot_half_heads(q) * sq_ref[...]
    k = k * ck_ref[...] + _rot_half_heads(k) * sk_ref[...]
    q_ref[...] = q.astype(jnp.bfloat16)
    k_ref[...] = k.astype(jnp.bfloat16)
    v_ref[...] = v.astype(jnp.bfloat16)


def _qkv(x, ln1_w, wq, wk, wv, cos_q, sin_q, cos_k, sin_k):
    return pl.pallas_call(
        _qkv_kernel,
        grid=(NSB,),
        in_specs=[
            pl.BlockSpec((SB, D), lambda i: (i, 0)),
            pl.BlockSpec((1, D), lambda i: (0, 0)),
            pl.BlockSpec((D, H * HD), lambda i: (0, 0)),
            pl.BlockSpec((D, KVH * HD), lambda i: (0, 0)),
            pl.BlockSpec((D, KVH * HD), lambda i: (0, 0)),
            pl.BlockSpec((SB, H * HD), lambda i: (i, 0)),
            pl.BlockSpec((SB, H * HD), lambda i: (i, 0)),
            pl.BlockSpec((SB, KVH * HD), lambda i: (i, 0)),
            pl.BlockSpec((SB, KVH * HD), lambda i: (i, 0)),
        ],
        out_specs=[
            pl.BlockSpec((SB, H * HD), lambda i: (i, 0)),
            pl.BlockSpec((SB, KVH * HD), lambda i: (i, 0)),
            pl.BlockSpec((SB, KVH * HD), lambda i: (i, 0)),
        ],
        out_shape=[
            jax.ShapeDtypeStruct((S, H * HD), jnp.bfloat16),
            jax.ShapeDtypeStruct((S, KVH * HD), jnp.bfloat16),
            jax.ShapeDtypeStruct((S, KVH * HD), jnp.bfloat16),
        ],
    )(x, ln1_w, wq, wk, wv, cos_q, sin_q, cos_k, sin_k)


MQ = REP * BQ


def _attn_kernel(q_ref, k_ref, v_ref, o_ref, acc_ref, m_ref, l_ref):
    qb = pl.program_id(1)
    kb = pl.program_id(2)

    @pl.when(kb == 0)
    def _init():
        acc_ref[...] = jnp.zeros_like(acc_ref)
        m_ref[...] = jnp.full_like(m_ref, NEG)
        l_ref[...] = jnp.zeros_like(l_ref)

    @pl.when(kb * BK <= qb * BQ + BQ - 1)
    def _compute():
        q = q_ref[0, 0]
        k = k_ref[0]
        s = jax.lax.dot_general(q, k, (((1,), (1,)), ((), ())),
                                preferred_element_type=jnp.float32)
        s = s * (1.0 / math.sqrt(HD))
        r = jax.lax.broadcasted_iota(jnp.int32, (MQ, BK), 0) % BQ
        row = qb * BQ + r
        col = kb * BK + jax.lax.broadcasted_iota(jnp.int32, (MQ, BK), 1)
        s = jnp.where(row >= col, s, NEG)
        m_prev = m_ref[...]
        m_cur = jnp.max(s, axis=-1, keepdims=True)
        m_new = jnp.maximum(m_prev, m_cur)
        p = jnp.exp(s - m_new)
        alpha = jnp.exp(m_prev - m_new)
        l_ref[...] = l_ref[...] * alpha + jnp.sum(p, axis=-1, keepdims=True)
        acc_ref[...] = acc_ref[...] * alpha + jnp.dot(
            p.astype(jnp.bfloat16), v_ref[0],
            preferred_element_type=jnp.float32)
        m_ref[...] = m_new

    @pl.when(kb == pl.num_programs(2) - 1)
    def _final():
        o_ref[0, 0] = (acc_ref[...] / l_ref[...]).astype(jnp.bfloat16)


def _attention(q, k, v):
    nqb = S // BQ
    return pl.pallas_call(
        _attn_kernel,
        grid=(KVH, nqb, S // BK),
        in_specs=[
            pl.BlockSpec((1, 1, MQ, HD), lambda g, i, j: (g, i, 0, 0)),
            pl.BlockSpec((1, BK, HD), lambda g, i, j: (g, j, 0)),
            pl.BlockSpec((1, BK, HD), lambda g, i, j: (g, j, 0)),
        ],
        out_specs=pl.BlockSpec((1, 1, MQ, HD), lambda g, i, j: (g, i, 0, 0)),
        out_shape=jax.ShapeDtypeStruct((KVH, nqb, MQ, HD), jnp.bfloat16),
        scratch_shapes=[
            pltpu.VMEM((MQ, HD), jnp.float32),
            pltpu.VMEM((MQ, 1), jnp.float32),
            pltpu.VMEM((MQ, 1), jnp.float32),
        ],
    )(q, k, v)


def _post_kernel(x_ref, attn_ref, wo_ref, ln2_ref, gate_ref,
                 hid_ref, x2_ref, sel_ref, rw_ref, cnt_ref):
    ao = jnp.dot(attn_ref[...], wo_ref[...],
                 preferred_element_type=jnp.float32)
    hid = x_ref[...] + ao
    hid_ref[...] = hid
    x2 = _rms(hid, ln2_ref[...])
    x2_ref[...] = x2
    logits = jax.lax.dot_general(
        x2, gate_ref[...], (((1,), (0,)), ((), ())),
        preferred_element_type=jnp.float32,
        precision=jax.lax.Precision.HIGHEST)
    mx = jnp.max(logits, axis=-1, keepdims=True)
    p = jnp.exp(logits - mx)
    p = p / jnp.sum(p, axis=-1, keepdims=True)
    lane = jax.lax.broadcasted_iota(jnp.int32, p.shape, 1)
    m1 = jnp.max(p, axis=-1, keepdims=True)
    e1 = jnp.min(jnp.where(p == m1, lane, E), axis=-1, keepdims=True)
    p2 = jnp.where(lane == e1, -1.0, p)
    m2 = jnp.max(p2, axis=-1, keepdims=True)
    e2 = jnp.min(jnp.where(p2 == m2, lane, E), axis=-1, keepdims=True)
    denom = m1 + m2
    sel_ref[...] = jnp.concatenate([e1, e2], axis=1)
    rw_ref[...] = jnp.concatenate([m1 / denom, m2 / denom], axis=1)
    lane16 = jax.lax.broadcasted_iota(jnp.int32, (SB, LANES), 1)
    oh = ((lane16 == e1).astype(jnp.bfloat16)
          + (lane16 == e2).astype(jnp.bfloat16))
    nch = SB // TPW
    ar = jax.lax.broadcasted_iota(jnp.int32, (nch, SB), 0)
    ac = jax.lax.broadcasted_iota(jnp.int32, (nch, SB), 1) // TPW
    seg = (ar == ac).astype(jnp.bfloat16)
    cnt_ref[...] = jnp.dot(seg, oh,
                           preferred_element_type=jnp.float32).astype(jnp.int32)


def _post_attn(x, attn, wo, ln2_w, gate_w):
    nch = SB // TPW
    return pl.pallas_call(
        _post_kernel,
        grid=(NSB,),
        in_specs=[
            pl.BlockSpec((SB, D), lambda i: (i, 0)),
            pl.BlockSpec((SB, H * HD), lambda i: (i, 0)),
            pl.BlockSpec((H * HD, D), lambda i: (0, 0)),
            pl.BlockSpec((1, D), lambda i: (0, 0)),
            pl.BlockSpec((D, E), lambda i: (0, 0)),
        ],
        out_specs=[
            pl.BlockSpec((SB, D), lambda i: (i, 0)),
            pl.BlockSpec((SB, D), lambda i: (i, 0)),
            pl.BlockSpec((SB, TOPK), lambda i: (i, 0)),
            pl.BlockSpec((SB, TOPK), lambda i: (i, 0)),
            pl.BlockSpec((nch, LANES), lambda i: (i, 0)),
        ],
        out_shape=[
            jax.ShapeDtypeStruct((S, D), jnp.float32),
            jax.ShapeDtypeStruct((S, D), jnp.float32),
            jax.ShapeDtypeStruct((S, TOPK), jnp.int32),
            jax.ShapeDtypeStruct((S, TOPK), jnp.float32),
            jax.ShapeDtypeStruct((NW, LANES), jnp.int32),
        ],
    )(x, attn, wo, ln2_w, gate_w)


def _bc(vec, lane):
    idx = jnp.zeros((LANES,), jnp.int32) + lane
    return vec.at[idx].get(mode="promise_in_bounds")


def _dispatch_body(sel_hbm, cnt_hbm, x2_hbm,
                   xs_hbm, inv_hbm, bexp_hbm, nrows_hbm,
                   selv, k0v, k1v, cntv, t1v, t2v, x2rows, sem):
    nc = 2
    wid = lax.axis_index("s") * nc + lax.axis_index("c")
    iota = jnp.arange(LANES, dtype=jnp.int32)
    zero = jnp.zeros((LANES,), jnp.int32)
    widv = zero + wid

    pltpu.sync_copy(cnt_hbm, cntv)
    tot = jnp.zeros((LANES,), jnp.int32)
    pfx = jnp.zeros((LANES,), jnp.int32)
    for ch in range(NW):
        row = cntv[ch]
        tot = tot + row
        pfx = pfx + jnp.where((zero + ch) < widv, row, 0)
    nb = (tot + (BLK - 1)) // BLK
    blk_end = plsc.cumsum(nb)
    blk_start = blk_end - nb
    roff = blk_start * BLK + pfx

    pltpu.sync_copy(sel_hbm.at[pl.ds(wid * TPW, TPW)],
                    selv.at[pl.ds(0, TPW)])
    pltpu.sync_copy(sel_hbm.at[pl.ds(S + wid * TPW, TPW)],
                    selv.at[pl.ds(TPW, TPW)])
    offs = [_bc(roff, e) for e in range(E)]
    runs = [jnp.zeros((LANES,), jnp.int32) for _ in range(E)]
    nchunk = PPW // LANES
    for c in range(nchunk):
        v = selv[pl.ds(c * LANES, LANES)]
        dest = jnp.zeros((LANES,), jnp.int32)
        for e in range(E):
            m = v == e
            csum = plsc.cumsum(jnp.where(m, 1, 0))
            dest = jnp.where(m, offs[e] + runs[e] + csum - 1, dest)
            runs[e] = runs[e] + plsc.all_reduce_population_count(m)
        if c < nchunk // 2:
            k0v[pl.ds(c * LANES, LANES)] = dest
        else:
            k1v[pl.ds((c - nchunk // 2) * LANES, LANES)] = dest
    pltpu.sync_copy(k0v, inv_hbm.at[pl.ds(wid * TPW, TPW)])
    pltpu.sync_copy(k1v, inv_hbm.at[pl.ds(S + wid * TPW, TPW)])

    pltpu.sync_copy(x2_hbm.at[pl.ds(wid * TPW, TPW)], x2rows)
    pltpu.async_copy(x2rows, xs_hbm.at[k0v], sem).wait()
    pltpu.async_copy(x2rows, xs_hbm.at[k1v], sem).wait()

    @pl.when(wid < NBP // LANES)
    def _tables():
        b_vec = zero + wid * LANES + iota
        be = jnp.zeros((LANES,), jnp.int32)
        for e in range(E):
            be = be + jnp.where(b_vec >= _bc(blk_end, e), 1, 0)
        t1v[...] = jnp.minimum(be, E - 1)
        nr = jnp.zeros((LANES,), jnp.int32)
        for e in range(E):
            rem = (_bc(tot, e) - (b_vec - _bc(blk_start, e)) * BLK)
            nr = jnp.where(be == e, jnp.clip(rem, 0, BLK), nr)
        t2v[...] = nr
        pltpu.sync_copy(t1v, bexp_hbm.at[pl.ds(wid * LANES, LANES)])
        pltpu.sync_copy(t2v, nrows_hbm.at[pl.ds(wid * LANES, LANES)])


def _dispatch(sel_flat, cnt, x2f):
    mesh = plsc.VectorSubcoreMesh(core_axis_name="c", subcore_axis_name="s")
    f = pl.kernel(
        _dispatch_body,
        out_type=[
            jax.ShapeDtypeStruct((PP, D), jnp.float32),
            jax.ShapeDtypeStruct((NP,), jnp.int32),
            jax.ShapeDtypeStruct((NBP,), jnp.int32),
            jax.ShapeDtypeStruct((NBP,), jnp.int32),
        ],
        mesh=mesh,
        compiler_params=pltpu.CompilerParams(needs_layout_passes=False),
        scratch_types=[
            pltpu.VMEM((PPW,), jnp.int32),
            pltpu.VMEM((TPW,), jnp.int32),
            pltpu.VMEM((TPW,), jnp.int32),
            pltpu.VMEM((NW, LANES), jnp.int32),
            pltpu.VMEM((LANES,), jnp.int32),
            pltpu.VMEM((LANES,), jnp.int32),
            pltpu.VMEM((TPW, D), jnp.float32),
            pltpu.SemaphoreType.DMA,
        ],
    )
    return f(sel_flat, cnt, x2f)


def _gmm_kernel(bexp_ref, nrows_ref, xs_ref, w1_ref, w3_ref, w2_ref, ys_ref):
    b = pl.program_id(0)
    n = nrows_ref[b]

    @pl.when(n > 0)
    def _compute():
        rowi = jax.lax.broadcasted_iota(jnp.int32, (BLK, D), 0)
        x = jnp.where(rowi < n, xs_ref[...], 0.0).astype(jnp.bfloat16)
        a = jnp.dot(x, w1_ref[0], preferred_element_type=jnp.float32)
        t = jnp.dot(x, w3_ref[0], preferred_element_type=jnp.float32)
        g = (a * jax.lax.logistic(a)) * t
        ys_ref[...] = jnp.dot(g.astype(jnp.bfloat16), w2_ref[0],
                              preferred_element_type=jnp.float32)


def _gmm(bexp, nrows, xs, w1, w3, w2):
    grid_spec = pltpu.PrefetchScalarGridSpec(
        num_scalar_prefetch=2,
        grid=(NB,),
        in_specs=[
            pl.BlockSpec((BLK, D), lambda b, be, nr: (b, 0)),
            pl.BlockSpec((1, D, F), lambda b, be, nr: (be[b], 0, 0)),
            pl.BlockSpec((1, D, F), lambda b, be, nr: (be[b], 0, 0)),
            pl.BlockSpec((1, F, D), lambda b, be, nr: (be[b], 0, 0)),
        ],
        out_specs=pl.BlockSpec((BLK, D), lambda b, be, nr: (b, 0)),
    )
    return pl.pallas_call(
        _gmm_kernel,
        grid_spec=grid_spec,
        out_shape=jax.ShapeDtypeStruct((PP, D), jnp.float32),
    )(bexp, nrows, xs, w1, w3, w2)


def _combine_body(hid_hbm, ys_hbm, inv_hbm, rw_hbm, out_hbm,
                  i0v, i1v, rw0v, rw1v, rows0, rows1, hidv, outv, sem):
    nc = 2
    wid = lax.axis_index("s") * nc + lax.axis_index("c")
    zero = jnp.zeros((LANES,), jnp.int32)
    half = TPW // 2
    for h in range(2):
        tbase = wid * TPW + h * half
        pltpu.sync_copy(inv_hbm.at[pl.ds(tbase, half)], i0v)
        pltpu.sync_copy(inv_hbm.at[pl.ds(S + tbase, half)], i1v)
        pltpu.async_copy(ys_hbm.at[i0v], rows0, sem).wait()
        pltpu.async_copy(ys_hbm.at[i1v], rows1, sem).wait()
        pltpu.sync_copy(rw_hbm.at[pl.ds(tbase, half)], rw0v)
        pltpu.sync_copy(rw_hbm.at[pl.ds(S + tbase, half)], rw1v)
        pltpu.sync_copy(hid_hbm.at[pl.ds(tbase, half)], hidv)

        def body(i, carry):
            w0 = plsc.load_gather(rw0v, [zero + i])
            w1 = plsc.load_gather(rw1v, [zero + i])
            for j in range(D // LANES):
                sl = pl.ds(j * LANES, LANES)
                outv[i, sl] = (hidv[i, sl] + w0 * rows0[i, sl]
                               + w1 * rows1[i, sl])
            return carry

        lax.fori_loop(0, half, body, 0)
        pltpu.sync_copy(outv, out_hbm.at[pl.ds(tbase, half)])


def _combine(hidden, ys, inv, rw_flat):
    mesh = plsc.VectorSubcoreMesh(core_axis_name="c", subcore_axis_name="s")
    half = TPW // 2
    f = pl.kernel(
        _combine_body,
        out_type=jax.ShapeDtypeStruct((S, D), jnp.float32),
        mesh=mesh,
        compiler_params=pltpu.CompilerParams(needs_layout_passes=False),
        scratch_types=[
            pltpu.VMEM((half,), jnp.int32),
            pltpu.VMEM((half,), jnp.int32),
            pltpu.VMEM((half,), jnp.float32),
            pltpu.VMEM((half,), jnp.float32),
            pltpu.VMEM((half, D), jnp.float32),
            pltpu.VMEM((half, D), jnp.float32),
            pltpu.VMEM((half, D), jnp.float32),
            pltpu.VMEM((half, D), jnp.float32),
            pltpu.SemaphoreType.DMA,
        ],
    )
    return f(hidden, ys, inv, rw_flat)


def kernel(hidden_states, attention_mask, position_ids, ln1_w, ln2_w,
           wq, wk, wv, wo, gate_w, w1, w3, w2):
    x = hidden_states.reshape(S, D)

    inv_freq = 1.0 / (THETA ** (jnp.arange(0, HD, 2, dtype=jnp.float32) / HD))
    pos = position_ids.reshape(S).astype(jnp.float32)
    freqs = pos[:, None] * inv_freq[None, :]
    emb = jnp.concatenate([freqs, freqs], axis=-1)
    cos, sin = jnp.cos(emb), jnp.sin(emb)
    cos_q = jnp.tile(cos, (1, H))
    sin_q = jnp.tile(sin, (1, H))
    cos_k = jnp.tile(cos, (1, KVH))
    sin_k = jnp.tile(sin, (1, KVH))

    wq_b = wq.astype(jnp.bfloat16)
    wk_b = wk.astype(jnp.bfloat16)
    wv_b = wv.astype(jnp.bfloat16)
    wo_b = wo.astype(jnp.bfloat16)
    w1_b = w1.astype(jnp.bfloat16)
    w3_b = w3.astype(jnp.bfloat16)
    w2_b = w2.astype(jnp.bfloat16)

    q, k, v = _qkv(x, ln1_w.reshape(1, D), wq_b, wk_b, wv_b,
                   cos_q, sin_q, cos_k, sin_k)

    nqb = S // BQ
    qh = (q.reshape(nqb, BQ, KVH, REP, HD)
          .transpose(2, 0, 3, 1, 4).reshape(KVH, nqb, MQ, HD))
    kh = k.reshape(S, KVH, HD).transpose(1, 0, 2)
    vh = v.reshape(S, KVH, HD).transpose(1, 0, 2)
    attn = _attention(qh, kh, vh)
    attn2 = (attn.reshape(KVH, nqb, REP, BQ, HD)
             .transpose(1, 3, 0, 2, 4).reshape(S, H * HD))

    hidden, x2f, sel, rw, cnt = _post_attn(x, attn2, wo_b,
                                           ln2_w.reshape(1, D), gate_w)

    sel_flat = sel.T.reshape(NP)
    rw_flat = rw.T.reshape(NP)
    xs, inv, bexp, nrows = _dispatch(sel_flat, cnt, x2f)
    ys = _gmm(bexp, nrows, xs, w1_b, w3_b, w2_b)
    out = _combine(hidden, ys, inv, rw_flat)
    return out.reshape(B, S, D)
